# speedup vs baseline: 1.0203x; 1.0203x over previous
import functools

import jax
import jax.numpy as jnp
from jax.experimental import pallas as pl

H = 128
N_HERB = 2048


def _matmul(x, w, bm=256, bk=512):
    M, K = x.shape
    K2, Ho = w.shape
    assert K == K2
    bk = min(bk, K)
    assert M % bm == 0 and K % bk == 0
    grid = (M // bm, K // bk)

    def body(x_ref, w_ref, o_ref):
        k = pl.program_id(1)

        @pl.when(k == 0)
        def _():
            o_ref[...] = jnp.zeros_like(o_ref)

        o_ref[...] += jnp.dot(x_ref[...], w_ref[...],
                              preferred_element_type=jnp.float32)

    return pl.pallas_call(
        body,
        grid=grid,
        in_specs=[pl.BlockSpec((bm, bk), lambda i, k: (i, k)),
                  pl.BlockSpec((bk, Ho), lambda i, k: (k, 0))],
        out_specs=pl.BlockSpec((bm, Ho), lambda i, k: (i, 0)),
        out_shape=jax.ShapeDtypeStruct((M, Ho), jnp.float32),
    )(x, w)


def _gcn(x, src, dst, norm, W, b, n):
    xw = _matmul(x, W)
    out = jnp.zeros((n, H), jnp.float32).at[dst].add(xw[src] * norm[:, None])
    return out + b


def _gat(x, src, dst, W, a_s, a_d, b, n):
    heads, c = a_s.shape
    xw = _matmul(x, W).reshape(n, heads, c)
    al_s = (xw * a_s[None]).sum(-1)
    al_d = (xw * a_d[None]).sum(-1)
    alpha = jax.nn.leaky_relu(al_s[src] + al_d[dst], negative_slope=0.2)
    amax = jax.ops.segment_max(alpha, dst, num_segments=n)
    ex = jnp.exp(alpha - amax[dst])
    den = jax.ops.segment_sum(ex, dst, num_segments=n)
    att = ex / (den[dst] + 1e-16)
    out = jax.ops.segment_sum(xw[src] * att[:, :, None], dst, num_segments=n)
    return out.mean(axis=1) + b


def _branch(x, ei, p, pre, n):
    h0 = _matmul(x, p[pre + "_proj_W"]) + p[pre + "_proj_b"]
    loop = jnp.arange(n, dtype=ei.dtype)
    src = jnp.concatenate([ei[0], loop])
    dst = jnp.concatenate([ei[1], loop])
    deg = jnp.zeros((n,), jnp.float32).at[dst].add(1.0)
    dis = jnp.where(deg > 0, jax.lax.rsqrt(deg), 0.0)
    norm = dis[src] * dis[dst]
    g1 = jax.nn.relu(_gcn(h0, src, dst, norm, p[pre + "_gcn1_W"],
                          p[pre + "_gcn1_b"], n) + h0)
    ga = jax.nn.relu(_gat(g1, src, dst, p[pre + "_gat_W"], p[pre + "_gat_as"],
                          p[pre + "_gat_ad"], p[pre + "_gat_b"], n) + g1)
    g2 = jax.nn.relu(_gcn(ga, src, dst, norm, p[pre + "_gcn2_W"],
                          p[pre + "_gcn2_b"], n) + ga)
    W = p[pre + "_cnn_W"]
    return (_matmul(g1, W[:, 0, :].T) + _matmul(g2, W[:, 1, :].T)
            + p[pre + "_cnn_b"])


def kernel(het_net, het_x, herb_net, herb_x, target_net, target_x, params):
    p = params
    het_e = _branch(het_x, het_net, p, "het", het_x.shape[0])
    herb_e = _branch(herb_x, herb_net, p, "herb", herb_x.shape[0])
    tgt_e = _branch(target_x, target_net, p, "tgt", target_x.shape[0])
    C = het_e[:N_HERB]
    D = het_e[N_HERB:]
    return _matmul(C, D.T) + _matmul(herb_e, tgt_e.T)


# speedup vs baseline: 5.4449x; 5.3365x over previous
import dataclasses
import functools

import jax
import jax.numpy as jnp
from jax import lax
from jax.experimental import pallas as pl
from jax.experimental.pallas import tpu as pltpu
from jax.experimental.pallas import tpu_sc as plsc

H = 128
N_HERB = 2048
NC = 2
NS = 16
NT = NC * NS
LANES = 16


def _mesh():
    return plsc.VectorSubcoreMesh(core_axis_name="c", subcore_axis_name="s")


def _sc_params():
    cp = pltpu.CompilerParams()
    fields = pltpu.CompilerParams.__dataclass_fields__
    if "needs_layout_passes" in fields:
        cp = dataclasses.replace(cp, needs_layout_passes=False)
    if "use_tc_tiling_on_sc" in fields:
        cp = dataclasses.replace(cp, use_tc_tiling_on_sc=False)
    return cp


def _zero_rows(buf):
    R, W = buf.shape
    z = jnp.zeros((LANES,), jnp.float32)

    @pl.loop(0, R)
    def _(i):
        buf[i, :] = z


def _sc_hist(dst, npad):
    (E,) = dst.shape
    CH = 128
    cpt = E // (NT * CH)

    @functools.partial(
        pl.kernel,
        out_type=jax.ShapeDtypeStruct((npad, NT * LANES), jnp.float32),
        mesh=_mesh(),
        compiler_params=_sc_params(),
        scratch_types=[
            pltpu.VMEM((CH,), jnp.int32),
            pltpu.VMEM((npad, LANES), jnp.float32),
        ],
    )
    def k(dst_hbm, out_hbm, d_v, acc_v):
        cid = lax.axis_index("c")
        sid = lax.axis_index("s")
        t = sid * NC + cid
        _zero_rows(acc_v)
        iota = lax.iota(jnp.int32, LANES)
        ones = jnp.full((LANES,), 1.0, jnp.float32)

        @pl.loop(0, cpt)
        def _(j):
            base = (t * cpt + j) * CH
            pltpu.sync_copy(dst_hbm.at[pl.ds(base, CH)], d_v)

            @pl.loop(0, CH)
            def _(e):
                dsp = plsc.load_gather(d_v, [jnp.full((LANES,), e,
                                                      jnp.int32)])
                plsc.addupdate_scatter(acc_v, [dsp, iota], ones)

        pltpu.sync_copy(acc_v, out_hbm.at[:, pl.ds(t * LANES, LANES)])

    return k(dst)


def _sc_gs(tab8, src, dst, npad):
    (E,) = src.shape
    CH = 128
    NG = 4
    cpt = E // (NG * CH)

    @functools.partial(
        pl.kernel,
        out_type=jax.ShapeDtypeStruct((NG, npad, H), jnp.float32),
        mesh=_mesh(),
        compiler_params=_sc_params(),
        scratch_types=[
            pltpu.VMEM((CH,), jnp.int32),
            pltpu.VMEM((CH,), jnp.int32),
            pltpu.VMEM((CH,), jnp.int32),
            pltpu.VMEM((CH, LANES), jnp.float32),
            pltpu.VMEM((npad, LANES), jnp.float32),
            pltpu.SemaphoreType.DMA,
        ],
    )
    def k(tab_hbm, src_hbm, dst_hbm, out_hbm, s_v, d_v, gi_v, rows_v,
          acc_v, sem):
        cid = lax.axis_index("c")
        sid = lax.axis_index("s")
        t = sid * NC + cid
        g = t // 8
        cs = t - g * 8
        _zero_rows(acc_v)
        iota = lax.iota(jnp.int32, LANES)

        @pl.loop(0, cpt)
        def _(j):
            base = (g * cpt + j) * CH
            pltpu.sync_copy(src_hbm.at[pl.ds(base, CH)], s_v)
            pltpu.sync_copy(dst_hbm.at[pl.ds(base, CH)], d_v)

            @pl.loop(0, CH, step=LANES)
            def _(o):
                gi_v[pl.ds(o, LANES)] = s_v[pl.ds(o, LANES)] * 8 + cs

            pltpu.async_copy(tab_hbm.at[gi_v], rows_v, sem).wait()

            @pl.loop(0, CH)
            def _(e):
                dsp = plsc.load_gather(d_v, [jnp.full((LANES,), e,
                                                      jnp.int32)])
                plsc.addupdate_scatter(acc_v, [dsp, iota], rows_v[e, :])

        pltpu.sync_copy(acc_v, out_hbm.at[g, :, pl.ds(cs * LANES, LANES)])

    return k(tab8, src, dst)


def _sc_den(src, dst, als_t, ald_t, smax16, npad):
    (E,) = src.shape
    CH = 128
    G = CH // LANES
    NG = 8
    cpt = E // (NG * CH)

    @functools.partial(
        pl.kernel,
        out_type=(jax.ShapeDtypeStruct((4, E), jnp.float32),
                  jax.ShapeDtypeStruct((npad, NT * LANES), jnp.float32)),
        mesh=_mesh(),
        compiler_params=_sc_params(),
        scratch_types=[
            pltpu.VMEM((CH,), jnp.int32),
            pltpu.VMEM((CH,), jnp.int32),
            pltpu.VMEM((npad,), jnp.float32),
            pltpu.VMEM((npad,), jnp.float32),
            pltpu.VMEM((LANES,), jnp.float32),
            pltpu.VMEM((CH,), jnp.float32),
            pltpu.VMEM((npad, LANES), jnp.float32),
        ],
    )
    def k(src_hbm, dst_hbm, als_hbm, ald_hbm, sm_hbm, q_hbm, z_hbm,
          s_v, d_v, als_v, ald_v, sm_v, qb, acc_v):
        cid = lax.axis_index("c")
        sid = lax.axis_index("s")
        t = sid * NC + cid
        g = t // 4
        h = t - g * 4
        pltpu.sync_copy(als_hbm.at[h], als_v)
        pltpu.sync_copy(ald_hbm.at[h], ald_v)
        pltpu.sync_copy(sm_hbm, sm_v)
        _zero_rows(acc_v)
        iota = lax.iota(jnp.int32, LANES)
        smsp = plsc.load_gather(sm_v, [jnp.full((LANES,), h, jnp.int32)])

        @pl.loop(0, cpt)
        def _(j):
            base = (g * cpt + j) * CH
            pltpu.sync_copy(src_hbm.at[pl.ds(base, CH)], s_v)
            pltpu.sync_copy(dst_hbm.at[pl.ds(base, CH)], d_v)

            @pl.loop(0, G)
            def _(gg):
                sl = pl.ds(gg * LANES, LANES)
                av = plsc.load_gather(als_v, [s_v[sl]])
                dv = plsc.load_gather(ald_v, [d_v[sl]])
                al = av + dv
                al = jnp.maximum(al, al * 0.2)
                cc = dv + smsp
                cc = jnp.maximum(cc, cc * 0.2)
                qb[sl] = jnp.exp(al - cc)

            @pl.loop(0, CH)
            def _(e):
                ee = jnp.full((LANES,), e, jnp.int32)
                dsp = plsc.load_gather(d_v, [ee])
                qsp = plsc.load_gather(qb, [ee])
                plsc.addupdate_scatter(acc_v, [dsp, iota], qsp)

            pltpu.sync_copy(qb, q_hbm.at[h, pl.ds(base, CH)])

        pltpu.sync_copy(acc_v, z_hbm.at[:, pl.ds(t * LANES, LANES)])

    return k(src, dst, als_t, ald_t, smax16)


def _sc_att(q, dst, inv, npad):
    four, E = q.shape
    CH = 128
    G = CH // LANES
    cpt = E // (NT * CH)

    @functools.partial(
        pl.kernel,
        out_type=jax.ShapeDtypeStruct((4, E), jnp.float32),
        mesh=_mesh(),
        compiler_params=_sc_params(),
        scratch_types=[
            pltpu.VMEM((CH,), jnp.int32),
            pltpu.VMEM((CH,), jnp.float32),
            pltpu.VMEM((npad * 4,), jnp.float32),
        ],
    )
    def k(q_hbm, dst_hbm, inv_hbm, w_hbm, d_v, q_v, inv_v):
        cid = lax.axis_index("c")
        sid = lax.axis_index("s")
        t = sid * NC + cid
        pltpu.sync_copy(inv_hbm, inv_v)

        @pl.loop(0, cpt)
        def _(j):
            base = (t * cpt + j) * CH
            pltpu.sync_copy(dst_hbm.at[pl.ds(base, CH)], d_v)
            for h in range(4):
                pltpu.sync_copy(q_hbm.at[h, pl.ds(base, CH)], q_v)

                @pl.loop(0, G)
                def _(gg):
                    sl = pl.ds(gg * LANES, LANES)
                    iv = plsc.load_gather(inv_v, [d_v[sl] * 4 + h])
                    q_v[sl] = q_v[sl] * iv

                pltpu.sync_copy(q_v, w_hbm.at[h, pl.ds(base, CH)])

    return k(q, dst, inv)


def _sc_gsw(tab32, src, dst, w4, npad):
    (E,) = src.shape
    CH = 128
    NG = 4
    cpt = E // (NG * CH)

    @functools.partial(
        pl.kernel,
        out_type=jax.ShapeDtypeStruct((NG, npad, H), jnp.float32),
        mesh=_mesh(),
        compiler_params=_sc_params(),
        scratch_types=[
            pltpu.VMEM((CH,), jnp.int32),
            pltpu.VMEM((CH,), jnp.int32),
            pltpu.VMEM((CH,), jnp.int32),
            pltpu.VMEM((4 * CH,), jnp.float32),
            pltpu.VMEM((4, CH, LANES), jnp.float32),
            pltpu.VMEM((npad, LANES), jnp.float32),
            pltpu.SemaphoreType.DMA,
        ],
    )
    def k(tab_hbm, src_hbm, dst_hbm, w_hbm, out_hbm, s_v, d_v, gi_v, wb_v,
          rows_v, acc_v, sem):
        cid = lax.axis_index("c")
        sid = lax.axis_index("s")
        t = sid * NC + cid
        g = t // 8
        cs = t - g * 8
        _zero_rows(acc_v)
        iota = lax.iota(jnp.int32, LANES)

        @pl.loop(0, cpt)
        def _(j):
            base = (g * cpt + j) * CH
            pltpu.sync_copy(src_hbm.at[pl.ds(base, CH)], s_v)
            pltpu.sync_copy(dst_hbm.at[pl.ds(base, CH)], d_v)
            for h in range(4):
                pltpu.sync_copy(w_hbm.at[h, pl.ds(base, CH)],
                                wb_v.at[pl.ds(h * CH, CH)])

                @pl.loop(0, CH, step=LANES)
                def _(o):
                    gi_v[pl.ds(o, LANES)] = (s_v[pl.ds(o, LANES)] * 32
                                             + (h * 8) + cs)

                pltpu.async_copy(tab_hbm.at[gi_v], rows_v.at[h],
                                 sem).wait()

            @pl.loop(0, CH)
            def _(e):
                ee = jnp.full((LANES,), e, jnp.int32)
                dsp = plsc.load_gather(d_v, [ee])
                w0 = plsc.load_gather(wb_v, [ee])
                acc = rows_v[0, e, :] * w0
                for h in range(1, 4):
                    wh = plsc.load_gather(wb_v, [h * CH + ee])
                    acc = acc + rows_v[h, e, :] * wh
                plsc.addupdate_scatter(acc_v, [dsp, iota], acc)

        pltpu.sync_copy(acc_v, out_hbm.at[g, :, pl.ds(cs * LANES, LANES)])

    return k(tab32, src, dst, w4)


def _matmul_bias(x, w, b, bm=256, bk=512):
    M, K = x.shape
    _, Ho = w.shape
    bk = min(bk, K)
    grid = (M // bm, K // bk)

    def body(x_ref, w_ref, b_ref, o_ref):
        kk = pl.program_id(1)

        @pl.when(kk == 0)
        def _():
            o_ref[...] = jnp.broadcast_to(b_ref[...], o_ref.shape)

        o_ref[...] += jnp.dot(x_ref[...], w_ref[...],
                              preferred_element_type=jnp.float32)

    return pl.pallas_call(
        body,
        grid=grid,
        in_specs=[pl.BlockSpec((bm, bk), lambda i, k: (i, k)),
                  pl.BlockSpec((bk, Ho), lambda i, k: (k, 0)),
                  pl.BlockSpec((1, Ho), lambda i, k: (0, 0))],
        out_specs=pl.BlockSpec((bm, Ho), lambda i, k: (i, 0)),
        out_shape=jax.ShapeDtypeStruct((M, Ho), jnp.float32),
    )(x, w, b)


def _scale_matmul(x, w, s, bm=512):
    M, K = x.shape
    _, Ho = w.shape

    def body(x_ref, w_ref, s_ref, o_ref):
        o_ref[...] = jnp.dot(x_ref[...], w_ref[...],
                             preferred_element_type=jnp.float32) * s_ref[...]

    return pl.pallas_call(
        body,
        grid=(M // bm,),
        in_specs=[pl.BlockSpec((bm, K), lambda i: (i, 0)),
                  pl.BlockSpec((K, Ho), lambda i: (0, 0)),
                  pl.BlockSpec((bm, 1), lambda i: (i, 0))],
        out_specs=pl.BlockSpec((bm, Ho), lambda i: (i, 0)),
        out_shape=jax.ShapeDtypeStruct((M, Ho), jnp.float32),
    )(x, w, s)


def _gat_lin(g1, wg, a_s, a_d, bm=512):
    M, K = g1.shape
    _, W4 = wg.shape

    def body(g_ref, w_ref, as_ref, ad_ref, xw_ref, als_ref, ald_ref,
             sm_ref):
        i = pl.program_id(0)
        xw = jnp.dot(g_ref[...], w_ref[...],
                     preferred_element_type=jnp.float32)
        xw_ref[...] = xw
        als_cols = []
        ald_cols = []
        for h in range(4):
            sl = xw[:, h * H:(h + 1) * H]
            als_cols.append(jnp.sum(sl * as_ref[h:h + 1, :], axis=1,
                                    keepdims=True))
            ald_cols.append(jnp.sum(sl * ad_ref[h:h + 1, :], axis=1,
                                    keepdims=True))
        als_blk = jnp.concatenate(als_cols, axis=1)
        ald_blk = jnp.concatenate(ald_cols, axis=1)
        als_ref[...] = als_blk
        ald_ref[...] = ald_blk
        loc = jnp.max(als_blk, axis=0, keepdims=True)

        @pl.when(i == 0)
        def _():
            sm_ref[...] = loc

        @pl.when(i > 0)
        def _():
            sm_ref[...] = jnp.maximum(sm_ref[...], loc)

    return pl.pallas_call(
        body,
        grid=(M // bm,),
        in_specs=[pl.BlockSpec((bm, K), lambda i: (i, 0)),
                  pl.BlockSpec((K, W4), lambda i: (0, 0)),
                  pl.BlockSpec((4, H), lambda i: (0, 0)),
                  pl.BlockSpec((4, H), lambda i: (0, 0))],
        out_specs=[pl.BlockSpec((bm, W4), lambda i: (i, 0)),
                   pl.BlockSpec((bm, 4), lambda i: (i, 0)),
                   pl.BlockSpec((bm, 4), lambda i: (i, 0)),
                   pl.BlockSpec((1, 4), lambda i: (0, 0))],
        out_shape=[jax.ShapeDtypeStruct((M, W4), jnp.float32),
                   jax.ShapeDtypeStruct((M, 4), jnp.float32),
                   jax.ShapeDtypeStruct((M, 4), jnp.float32),
                   jax.ShapeDtypeStruct((1, 4), jnp.float32)],
    )(g1, wg, a_s, a_d)


def _dis_kernel(degs):
    npad = degs.shape[0]

    def body(p_ref, o_ref):
        deg = jnp.sum(p_ref[...], axis=1, keepdims=True) * (1.0 / LANES)
        o_ref[...] = jnp.where(deg > 0, lax.rsqrt(deg), 0.0)

    return pl.pallas_call(
        body,
        out_shape=jax.ShapeDtypeStruct((npad, 1), jnp.float32),
    )(degs)


def _postden(zout, sel):
    npad = zout.shape[0]

    def body(p_ref, s_ref, o_ref):
        z = jnp.dot(p_ref[...], s_ref[...],
                    preferred_element_type=jnp.float32)
        o_ref[...] = 0.25 / (z + 1e-16)

    return pl.pallas_call(
        body,
        out_shape=jax.ShapeDtypeStruct((npad, 4), jnp.float32),
    )(zout, sel)


def _combine_gs(parts, dis, b, res, n):
    def body(p_ref, d_ref, b_ref, r_ref, o_ref):
        x = jnp.sum(p_ref[...], axis=0)[:n]
        o_ref[...] = jnp.maximum(x * d_ref[...] + b_ref[...] + r_ref[...],
                                 0.0)

    return pl.pallas_call(
        body,
        out_shape=jax.ShapeDtypeStruct((n, H), jnp.float32),
    )(parts, dis, b, res)


def _combine_gat(parts, b, res, n):
    def body(p_ref, b_ref, r_ref, o_ref):
        x = jnp.sum(p_ref[...], axis=0)[:n]
        o_ref[...] = jnp.maximum(x + b_ref[...] + r_ref[...], 0.0)

    return pl.pallas_call(
        body,
        out_shape=jax.ShapeDtypeStruct((n, H), jnp.float32),
    )(parts, b, res)


def _cnn(g1, g2, wc, b):
    M = g1.shape[0]
    dn = (((1,), (1,)), ((), ()))

    def body(g1_ref, g2_ref, w_ref, b_ref, o_ref):
        w0 = w_ref[:, 0, :]
        w1 = w_ref[:, 1, :]
        o_ref[...] = (lax.dot_general(g1_ref[...], w0, dn,
                                      preferred_element_type=jnp.float32)
                      + lax.dot_general(g2_ref[...], w1, dn,
                                        preferred_element_type=jnp.float32)
                      + b_ref[...])

    return pl.pallas_call(
        body,
        out_shape=jax.ShapeDtypeStruct((M, H), jnp.float32),
    )(g1, g2, wc, b)


def _final(C, D, he, te, bm=256, bn=512):
    M = C.shape[0]
    N2 = D.shape[0]
    dn = (((1,), (1,)), ((), ()))

    def body(c_ref, d_ref, h_ref, t_ref, o_ref):
        o_ref[...] = (lax.dot_general(c_ref[...], d_ref[...], dn,
                                      preferred_element_type=jnp.float32)
                      + lax.dot_general(h_ref[...], t_ref[...], dn,
                                        preferred_element_type=jnp.float32))

    return pl.pallas_call(
        body,
        grid=(M // bm, N2 // bn),
        in_specs=[pl.BlockSpec((bm, H), lambda i, j: (i, 0)),
                  pl.BlockSpec((bn, H), lambda i, j: (j, 0)),
                  pl.BlockSpec((bm, H), lambda i, j: (i, 0)),
                  pl.BlockSpec((bn, H), lambda i, j: (j, 0))],
        out_specs=pl.BlockSpec((bm, bn), lambda i, j: (i, j)),
        out_shape=jax.ShapeDtypeStruct((M, N2), jnp.float32),
    )(C, D, he, te)


def _branch(x, ei, p, pre):
    n = x.shape[0]
    npad = n + 128
    e = ei.shape[1]
    ep = e + n
    epad = -(-ep // 4096) * 4096
    loop = jnp.arange(n, dtype=jnp.int32)
    src = jnp.concatenate([ei[0].astype(jnp.int32), loop,
                           jnp.zeros((epad - ep,), jnp.int32)])
    dst = jnp.concatenate([ei[1].astype(jnp.int32), loop,
                           jnp.full((epad - ep,), n, jnp.int32)])

    degs = _sc_hist(dst, npad)
    dis = _dis_kernel(degs)
    disn = dis[:n]

    h0 = _matmul_bias(x, p[pre + "_proj_W"], p[pre + "_proj_b"][None])

    t1 = _scale_matmul(h0, p[pre + "_gcn1_W"], disn)
    a1 = _sc_gs(t1.reshape(n * 8, 16), src, dst, npad)
    g1 = _combine_gs(a1, disn, p[pre + "_gcn1_b"][None], h0, n)

    xw4, als, ald, smax = _gat_lin(g1, p[pre + "_gat_W"],
                                   p[pre + "_gat_as"], p[pre + "_gat_ad"])
    rowpad = jnp.zeros((4, npad - n), jnp.float32)
    als_t = jnp.concatenate([als.T, rowpad], axis=1)
    ald_t = jnp.concatenate([ald.T, rowpad], axis=1)
    smax16 = jnp.pad(smax.reshape(-1), (0, LANES - 4))
    q, zout = _sc_den(src, dst, als_t, ald_t, smax16, npad)
    sel = ((jnp.arange(NT * LANES)[:, None] // LANES) % 4
           == jnp.arange(4)[None, :]).astype(jnp.float32) / LANES
    inv = _postden(zout, sel)
    w4 = _sc_att(q, dst, inv.reshape(-1), npad)
    a2 = _sc_gsw(xw4.reshape(n * 32, 16), src, dst, w4, npad)
    ga = _combine_gat(a2, p[pre + "_gat_b"][None], g1, n)

    t2 = _scale_matmul(ga, p[pre + "_gcn2_W"], disn)
    a3 = _sc_gs(t2.reshape(n * 8, 16), src, dst, npad)
    g2 = _combine_gs(a3, disn, p[pre + "_gcn2_b"][None], ga, n)

    return _cnn(g1, g2, p[pre + "_cnn_W"], p[pre + "_cnn_b"][None])


def kernel(het_net, het_x, herb_net, herb_x, target_net, target_x, params):
    p = params
    het_e = _branch(het_x, het_net, p, "het")
    herb_e = _branch(herb_x, herb_net, p, "herb")
    tgt_e = _branch(target_x, target_net, p, "tgt")
    C = het_e[:N_HERB]
    D = het_e[N_HERB:]
    return _final(C, D, herb_e, tgt_e)


# speedup vs baseline: 10.3194x; 1.8952x over previous
import dataclasses
import functools

import jax
import jax.numpy as jnp
from jax import lax
from jax.experimental import pallas as pl
from jax.experimental.pallas import tpu as pltpu
from jax.experimental.pallas import tpu_sc as plsc

H = 128
N_HERB = 2048
NC = 2
NS = 16
NT = NC * NS
LANES = 16


def _mesh():
    return plsc.VectorSubcoreMesh(core_axis_name="c", subcore_axis_name="s")


def _sc_params():
    cp = pltpu.CompilerParams()
    fields = pltpu.CompilerParams.__dataclass_fields__
    if "needs_layout_passes" in fields:
        cp = dataclasses.replace(cp, needs_layout_passes=False)
    if "use_tc_tiling_on_sc" in fields:
        cp = dataclasses.replace(cp, use_tc_tiling_on_sc=False)
    return cp


def _zero_rows(buf):
    R, W = buf.shape
    z = jnp.zeros((LANES,), jnp.float32)

    @pl.loop(0, R)
    def _(i):
        buf[i, :] = z


def _sc_hist(dst, npad):
    (E,) = dst.shape
    CH = 128
    cpt = E // (NT * CH)

    @functools.partial(
        pl.kernel,
        out_type=jax.ShapeDtypeStruct((npad, NT * LANES), jnp.float32),
        mesh=_mesh(),
        compiler_params=_sc_params(),
        scratch_types=[
            pltpu.VMEM((CH,), jnp.int32),
            pltpu.VMEM((npad, LANES), jnp.float32),
        ],
    )
    def k(dst_hbm, out_hbm, d_v, acc_v):
        cid = lax.axis_index("c")
        sid = lax.axis_index("s")
        t = sid * NC + cid
        _zero_rows(acc_v)
        iota = lax.iota(jnp.int32, LANES)
        ones = jnp.full((LANES,), 1.0, jnp.float32)

        @pl.loop(0, cpt)
        def _(j):
            base = (t * cpt + j) * CH
            pltpu.sync_copy(dst_hbm.at[pl.ds(base, CH)], d_v)

            @pl.loop(0, CH)
            def _(e):
                dsp = plsc.load_gather(d_v, [jnp.full((LANES,), e,
                                                      jnp.int32)])
                plsc.addupdate_scatter(acc_v, [dsp, iota], ones)

        pltpu.sync_copy(acc_v, out_hbm.at[:, pl.ds(t * LANES, LANES)])

    return k(dst)


def _sc_gs(tab8, sd, npad, E):
    CH = 256
    NG = 4
    cpt = E // (NG * CH)
    half = cpt // 2

    @functools.partial(
        pl.kernel,
        out_type=jax.ShapeDtypeStruct((NG, npad, H), jnp.float32),
        mesh=_mesh(),
        compiler_params=_sc_params(),
        scratch_types=[
            pltpu.VMEM((2 * CH,), jnp.int32),
            pltpu.VMEM((2 * CH,), jnp.int32),
            pltpu.VMEM((CH,), jnp.int32),
            pltpu.VMEM((CH,), jnp.int32),
            pltpu.VMEM((CH, LANES), jnp.float32),
            pltpu.VMEM((CH, LANES), jnp.float32),
            pltpu.VMEM((npad, LANES), jnp.float32),
            pltpu.SemaphoreType.DMA,
            pltpu.SemaphoreType.DMA,
        ],
    )
    def k(tab_hbm, sd_hbm, out_hbm, sd0, sd1, gi0, gi1, rw0, rw1, acc_v,
          sm0, sm1):
        cid = lax.axis_index("c")
        sid = lax.axis_index("s")
        t = sid * NC + cid
        g = t // 8
        cs = t - g * 8
        _zero_rows(acc_v)
        iota = lax.iota(jnp.int32, LANES)
        sdv = (sd0, sd1)
        giv = (gi0, gi1)
        rwv = (rw0, rw1)
        smv = (sm0, sm1)

        def load(b, j):
            base = (g * cpt + j) * CH
            pltpu.sync_copy(sd_hbm.at[pl.ds(base * 2, 2 * CH)], sdv[b])

            @pl.loop(0, CH, step=LANES)
            def _(o):
                giv[b][pl.ds(o, LANES)] = sdv[b][pl.ds(o, LANES)] * 8 + cs

            pltpu.make_async_copy(tab_hbm.at[giv[b]], rwv[b],
                                  smv[b]).start()

        def wait(b):
            pltpu.make_async_copy(tab_hbm.at[giv[b]], rwv[b],
                                  smv[b]).wait()

        def proc(b):
            @pl.loop(0, CH)
            def _(e):
                dsp = plsc.load_gather(
                    sdv[b], [jnp.full((LANES,), CH + e, jnp.int32)])
                plsc.addupdate_scatter(acc_v, [dsp, iota], rwv[b][e, :])

        load(0, 0)

        @pl.loop(0, half)
        def _(j2):
            j = j2 * 2
            load(1, j + 1)
            wait(0)
            proc(0)
            load(0, jnp.minimum(j + 2, cpt - 1))
            wait(1)
            proc(1)

        wait(0)
        pltpu.sync_copy(acc_v, out_hbm.at[g, :, pl.ds(cs * LANES, LANES)])

    return k(tab8, sd)


def _sc_den(sd, als_t, ald_t, smax16, npad, E):
    CH = 128
    G = CH // LANES
    NG = 8
    cpt = E // (NG * CH)

    @functools.partial(
        pl.kernel,
        out_type=(jax.ShapeDtypeStruct((4, E), jnp.float32),
                  jax.ShapeDtypeStruct((npad, NT * LANES), jnp.float32)),
        mesh=_mesh(),
        compiler_params=_sc_params(),
        scratch_types=[
            pltpu.VMEM((2 * CH,), jnp.int32),
            pltpu.VMEM((npad,), jnp.float32),
            pltpu.VMEM((npad,), jnp.float32),
            pltpu.VMEM((LANES,), jnp.float32),
            pltpu.VMEM((CH,), jnp.float32),
            pltpu.VMEM((npad, LANES), jnp.float32),
        ],
    )
    def k(sd_hbm, als_hbm, ald_hbm, sm_hbm, q_hbm, z_hbm,
          sd_v, als_v, ald_v, sm_v, qb, acc_v):
        cid = lax.axis_index("c")
        sid = lax.axis_index("s")
        t = sid * NC + cid
        g = t // 4
        h = t - g * 4
        pltpu.sync_copy(als_hbm.at[h], als_v)
        pltpu.sync_copy(ald_hbm.at[h], ald_v)
        pltpu.sync_copy(sm_hbm, sm_v)
        _zero_rows(acc_v)
        iota = lax.iota(jnp.int32, LANES)
        smsp = plsc.load_gather(sm_v, [jnp.full((LANES,), h, jnp.int32)])

        @pl.loop(0, cpt)
        def _(j):
            base = (g * cpt + j) * CH
            pltpu.sync_copy(sd_hbm.at[pl.ds(base * 2, 2 * CH)], sd_v)

            @pl.loop(0, G)
            def _(gg):
                sl = pl.ds(gg * LANES, LANES)
                av = plsc.load_gather(als_v, [sd_v[sl]])
                dv = plsc.load_gather(
                    ald_v, [sd_v[pl.ds(CH + gg * LANES, LANES)]])
                al = av + dv
                al = jnp.maximum(al, al * 0.2)
                cc = dv + smsp
                cc = jnp.maximum(cc, cc * 0.2)
                qb[sl] = jnp.exp(al - cc)

            @pl.loop(0, CH)
            def _(e):
                ee = jnp.full((LANES,), e, jnp.int32)
                dsp = plsc.load_gather(sd_v, [ee + CH])
                qsp = plsc.load_gather(qb, [ee])
                plsc.addupdate_scatter(acc_v, [dsp, iota], qsp)

            pltpu.sync_copy(qb, q_hbm.at[h, pl.ds(base, CH)])

        pltpu.sync_copy(acc_v, z_hbm.at[:, pl.ds(t * LANES, LANES)])

    return k(sd, als_t, ald_t, smax16)


def _sc_att(q, dst, inv, npad):
    four, E = q.shape
    CH = 128
    G = CH // LANES
    cpt = E // (NT * CH)

    @functools.partial(
        pl.kernel,
        out_type=jax.ShapeDtypeStruct((4, E), jnp.float32),
        mesh=_mesh(),
        compiler_params=_sc_params(),
        scratch_types=[
            pltpu.VMEM((CH,), jnp.int32),
            pltpu.VMEM((4, CH), jnp.float32),
            pltpu.VMEM((npad * 4,), jnp.float32),
            pltpu.SemaphoreType.DMA,
        ],
    )
    def k(q_hbm, dst_hbm, inv_hbm, w_hbm, d_v, q_v, inv_v, sem):
        cid = lax.axis_index("c")
        sid = lax.axis_index("s")
        t = sid * NC + cid
        pltpu.sync_copy(inv_hbm, inv_v)

        @pl.loop(0, cpt)
        def _(j):
            base = (t * cpt + j) * CH
            pltpu.sync_copy(dst_hbm.at[pl.ds(base, CH)], d_v)
            for h in range(4):
                pltpu.make_async_copy(q_hbm.at[h, pl.ds(base, CH)],
                                      q_v.at[h], sem).start()
            for h in range(4):
                pltpu.make_async_copy(q_hbm.at[h, pl.ds(base, CH)],
                                      q_v.at[h], sem).wait()
            for h in range(4):
                @pl.loop(0, G)
                def _(gg):
                    sl = pl.ds(gg * LANES, LANES)
                    iv = plsc.load_gather(inv_v, [d_v[sl] * 4 + h])
                    q_v[h, sl] = q_v[h, sl] * iv

            for h in range(4):
                pltpu.sync_copy(q_v.at[h], w_hbm.at[h, pl.ds(base, CH)])

    return k(q, dst, inv)


def _sc_gsw(tab32, sd, w4, npad, E):
    CH = 128
    NG = 4
    cpt = E // (NG * CH)
    half = cpt // 2

    @functools.partial(
        pl.kernel,
        out_type=jax.ShapeDtypeStruct((NG, npad, H), jnp.float32),
        mesh=_mesh(),
        compiler_params=_sc_params(),
        scratch_types=[
            pltpu.VMEM((2 * CH,), jnp.int32),
            pltpu.VMEM((2 * CH,), jnp.int32),
            pltpu.VMEM((4, CH), jnp.int32),
            pltpu.VMEM((4, CH), jnp.int32),
            pltpu.VMEM((4 * CH,), jnp.float32),
            pltpu.VMEM((4 * CH,), jnp.float32),
            pltpu.VMEM((4, CH, LANES), jnp.float32),
            pltpu.VMEM((4, CH, LANES), jnp.float32),
            pltpu.VMEM((npad, LANES), jnp.float32),
            pltpu.SemaphoreType.DMA,
            pltpu.SemaphoreType.DMA,
        ],
    )
    def k(tab_hbm, sd_hbm, w_hbm, out_hbm, sd0, sd1, gi0, gi1, wb0, wb1,
          rw0, rw1, acc_v, sm0, sm1):
        cid = lax.axis_index("c")
        sid = lax.axis_index("s")
        t = sid * NC + cid
        g = t // 8
        cs = t - g * 8
        _zero_rows(acc_v)
        iota = lax.iota(jnp.int32, LANES)
        sdv = (sd0, sd1)
        giv = (gi0, gi1)
        wbv = (wb0, wb1)
        rwv = (rw0, rw1)
        smv = (sm0, sm1)

        def load(b, j):
            base = (g * cpt + j) * CH
            pltpu.sync_copy(sd_hbm.at[pl.ds(base * 2, 2 * CH)], sdv[b])
            for h in range(4):
                @pl.loop(0, CH, step=LANES)
                def _(o):
                    giv[b][h, pl.ds(o, LANES)] = (
                        sdv[b][pl.ds(o, LANES)] * 32 + (h * 8) + cs)

                pltpu.make_async_copy(tab_hbm.at[giv[b].at[h]],
                                     rwv[b].at[h], smv[b]).start()
                pltpu.make_async_copy(w_hbm.at[h, pl.ds(base, CH)],
                                      wbv[b].at[pl.ds(h * CH, CH)],
                                      smv[b]).start()

        def wait(b):
            for h in range(4):
                pltpu.make_async_copy(tab_hbm.at[giv[b].at[h]],
                                      rwv[b].at[h], smv[b]).wait()
                pltpu.make_async_copy(w_hbm.at[h, pl.ds(0, CH)],
                                      wbv[b].at[pl.ds(h * CH, CH)],
                                      smv[b]).wait()

        def proc(b):
            @pl.loop(0, CH)
            def _(e):
                ee = jnp.full((LANES,), e, jnp.int32)
                dsp = plsc.load_gather(sdv[b], [ee + CH])
                w0 = plsc.load_gather(wbv[b], [ee])
                acc = rwv[b][0, e, :] * w0
                for h in range(1, 4):
                    wh = plsc.load_gather(wbv[b], [h * CH + ee])
                    acc = acc + rwv[b][h, e, :] * wh
                plsc.addupdate_scatter(acc_v, [dsp, iota], acc)

        load(0, 0)

        @pl.loop(0, half)
        def _(j2):
            j = j2 * 2
            load(1, j + 1)
            wait(0)
            proc(0)
            load(0, jnp.minimum(j + 2, cpt - 1))
            wait(1)
            proc(1)

        wait(0)
        pltpu.sync_copy(acc_v, out_hbm.at[g, :, pl.ds(cs * LANES, LANES)])

    return k(tab32, sd, w4)


def _matmul_bias(x, w, b, bm=256, bk=512):
    M, K = x.shape
    _, Ho = w.shape
    bk = min(bk, K)
    grid = (M // bm, K // bk)

    def body(x_ref, w_ref, b_ref, o_ref):
        kk = pl.program_id(1)

        @pl.when(kk == 0)
        def _():
            o_ref[...] = jnp.broadcast_to(b_ref[...], o_ref.shape)

        o_ref[...] += jnp.dot(x_ref[...], w_ref[...],
                              preferred_element_type=jnp.float32)

    return pl.pallas_call(
        body,
        grid=grid,
        in_specs=[pl.BlockSpec((bm, bk), lambda i, k: (i, k)),
                  pl.BlockSpec((bk, Ho), lambda i, k: (k, 0)),
                  pl.BlockSpec((1, Ho), lambda i, k: (0, 0))],
        out_specs=pl.BlockSpec((bm, Ho), lambda i, k: (i, 0)),
        out_shape=jax.ShapeDtypeStruct((M, Ho), jnp.float32),
    )(x, w, b)


def _scale_matmul(x, w, s, bm=512):
    M, K = x.shape
    _, Ho = w.shape

    def body(x_ref, w_ref, s_ref, o_ref):
        o_ref[...] = jnp.dot(x_ref[...], w_ref[...],
                             preferred_element_type=jnp.float32) * s_ref[...]

    return pl.pallas_call(
        body,
        grid=(M // bm,),
        in_specs=[pl.BlockSpec((bm, K), lambda i: (i, 0)),
                  pl.BlockSpec((K, Ho), lambda i: (0, 0)),
                  pl.BlockSpec((bm, 1), lambda i: (i, 0))],
        out_specs=pl.BlockSpec((bm, Ho), lambda i: (i, 0)),
        out_shape=jax.ShapeDtypeStruct((M, Ho), jnp.float32),
    )(x, w, s)


def _gat_lin(g1, wg, a_s, a_d, bm=512):
    M, K = g1.shape
    _, W4 = wg.shape

    def body(g_ref, w_ref, as_ref, ad_ref, xw_ref, als_ref, ald_ref,
             sm_ref):
        i = pl.program_id(0)
        xw = jnp.dot(g_ref[...], w_ref[...],
                     preferred_element_type=jnp.float32)
        xw_ref[...] = xw
        als_cols = []
        ald_cols = []
        for h in range(4):
            sl = xw[:, h * H:(h + 1) * H]
            als_cols.append(jnp.sum(sl * as_ref[h:h + 1, :], axis=1,
                                    keepdims=True))
            ald_cols.append(jnp.sum(sl * ad_ref[h:h + 1, :], axis=1,
                                    keepdims=True))
        als_blk = jnp.concatenate(als_cols, axis=1)
        ald_blk = jnp.concatenate(ald_cols, axis=1)
        als_ref[...] = als_blk
        ald_ref[...] = ald_blk
        loc = jnp.max(als_blk, axis=0, keepdims=True)

        @pl.when(i == 0)
        def _():
            sm_ref[...] = loc

        @pl.when(i > 0)
        def _():
            sm_ref[...] = jnp.maximum(sm_ref[...], loc)

    return pl.pallas_call(
        body,
        grid=(M // bm,),
        in_specs=[pl.BlockSpec((bm, K), lambda i: (i, 0)),
                  pl.BlockSpec((K, W4), lambda i: (0, 0)),
                  pl.BlockSpec((4, H), lambda i: (0, 0)),
                  pl.BlockSpec((4, H), lambda i: (0, 0))],
        out_specs=[pl.BlockSpec((bm, W4), lambda i: (i, 0)),
                   pl.BlockSpec((bm, 4), lambda i: (i, 0)),
                   pl.BlockSpec((bm, 4), lambda i: (i, 0)),
                   pl.BlockSpec((1, 4), lambda i: (0, 0))],
        out_shape=[jax.ShapeDtypeStruct((M, W4), jnp.float32),
                   jax.ShapeDtypeStruct((M, 4), jnp.float32),
                   jax.ShapeDtypeStruct((M, 4), jnp.float32),
                   jax.ShapeDtypeStruct((1, 4), jnp.float32)],
    )(g1, wg, a_s, a_d)


def _dis_kernel(degs):
    npad = degs.shape[0]

    def body(p_ref, o_ref):
        deg = jnp.sum(p_ref[...], axis=1, keepdims=True) * (1.0 / LANES)
        o_ref[...] = jnp.where(deg > 0, lax.rsqrt(deg), 0.0)

    return pl.pallas_call(
        body,
        out_shape=jax.ShapeDtypeStruct((npad, 1), jnp.float32),
    )(degs)


def _postden(zout, sel):
    npad = zout.shape[0]

    def body(p_ref, s_ref, o_ref):
        z = jnp.dot(p_ref[...], s_ref[...],
                    preferred_element_type=jnp.float32)
        o_ref[...] = 0.25 / (z + 1e-16)

    return pl.pallas_call(
        body,
        out_shape=jax.ShapeDtypeStruct((npad, 4), jnp.float32),
    )(zout, sel)


def _combine_gs(parts, dis, b, res, n):
    def body(p_ref, d_ref, b_ref, r_ref, o_ref):
        x = jnp.sum(p_ref[...], axis=0)[:n]
        o_ref[...] = jnp.maximum(x * d_ref[...] + b_ref[...] + r_ref[...],
                                 0.0)

    return pl.pallas_call(
        body,
        out_shape=jax.ShapeDtypeStruct((n, H), jnp.float32),
    )(parts, dis, b, res)


def _combine_gat(parts, b, res, n):
    def body(p_ref, b_ref, r_ref, o_ref):
        x = jnp.sum(p_ref[...], axis=0)[:n]
        o_ref[...] = jnp.maximum(x + b_ref[...] + r_ref[...], 0.0)

    return pl.pallas_call(
        body,
        out_shape=jax.ShapeDtypeStruct((n, H), jnp.float32),
    )(parts, b, res)


def _cnn(g1, g2, wc, b):
    M = g1.shape[0]
    dn = (((1,), (1,)), ((), ()))

    def body(g1_ref, g2_ref, w_ref, b_ref, o_ref):
        w0 = w_ref[:, 0, :]
        w1 = w_ref[:, 1, :]
        o_ref[...] = (lax.dot_general(g1_ref[...], w0, dn,
                                      preferred_element_type=jnp.float32)
                      + lax.dot_general(g2_ref[...], w1, dn,
                                        preferred_element_type=jnp.float32)
                      + b_ref[...])

    return pl.pallas_call(
        body,
        out_shape=jax.ShapeDtypeStruct((M, H), jnp.float32),
    )(g1, g2, wc, b)


def _final(C, D, he, te, bm=256, bn=512):
    M = C.shape[0]
    N2 = D.shape[0]
    dn = (((1,), (1,)), ((), ()))

    def body(c_ref, d_ref, h_ref, t_ref, o_ref):
        o_ref[...] = (lax.dot_general(c_ref[...], d_ref[...], dn,
                                      preferred_element_type=jnp.float32)
                      + lax.dot_general(h_ref[...], t_ref[...], dn,
                                        preferred_element_type=jnp.float32))

    return pl.pallas_call(
        body,
        grid=(M // bm, N2 // bn),
        in_specs=[pl.BlockSpec((bm, H), lambda i, j: (i, 0)),
                  pl.BlockSpec((bn, H), lambda i, j: (j, 0)),
                  pl.BlockSpec((bm, H), lambda i, j: (i, 0)),
                  pl.BlockSpec((bn, H), lambda i, j: (j, 0))],
        out_specs=pl.BlockSpec((bm, bn), lambda i, j: (i, j)),
        out_shape=jax.ShapeDtypeStruct((M, N2), jnp.float32),
    )(C, D, he, te)


def _pack_sd(src, dst, ch):
    return jnp.concatenate(
        [src.reshape(-1, 1, ch), dst.reshape(-1, 1, ch)],
        axis=1).reshape(-1)


def _branch(x, ei, p, pre):
    n = x.shape[0]
    npad = n + 128
    e = ei.shape[1]
    ep = e + n
    epad = -(-ep // 8192) * 8192
    loop = jnp.arange(n, dtype=jnp.int32)
    src = jnp.concatenate([ei[0].astype(jnp.int32), loop,
                           jnp.zeros((epad - ep,), jnp.int32)])
    dst = jnp.concatenate([ei[1].astype(jnp.int32), loop,
                           jnp.full((epad - ep,), n, jnp.int32)])
    sd_a = _pack_sd(src, dst, 256)
    sd_b = _pack_sd(src, dst, 128)

    degs = _sc_hist(dst, npad)
    dis = _dis_kernel(degs)
    disn = dis[:n]

    h0 = _matmul_bias(x, p[pre + "_proj_W"], p[pre + "_proj_b"][None])

    t1 = _scale_matmul(h0, p[pre + "_gcn1_W"], disn)
    a1 = _sc_gs(t1.reshape(n * 8, 16), sd_a, npad, epad)
    g1 = _combine_gs(a1, disn, p[pre + "_gcn1_b"][None], h0, n)

    xw4, als, ald, smax = _gat_lin(g1, p[pre + "_gat_W"],
                                   p[pre + "_gat_as"], p[pre + "_gat_ad"])
    rowpad = jnp.zeros((4, npad - n), jnp.float32)
    als_t = jnp.concatenate([als.T, rowpad], axis=1)
    ald_t = jnp.concatenate([ald.T, rowpad], axis=1)
    smax16 = jnp.pad(smax.reshape(-1), (0, LANES - 4))
    q, zout = _sc_den(sd_b, als_t, ald_t, smax16, npad, epad)
    sel = ((jnp.arange(NT * LANES)[:, None] // LANES) % 4
           == jnp.arange(4)[None, :]).astype(jnp.float32) / LANES
    inv = _postden(zout, sel)
    w4 = _sc_att(q, dst, inv.reshape(-1), npad)
    a2 = _sc_gsw(xw4.reshape(n * 32, 16), sd_b, w4, npad, epad)
    ga = _combine_gat(a2, p[pre + "_gat_b"][None], g1, n)

    t2 = _scale_matmul(ga, p[pre + "_gcn2_W"], disn)
    a3 = _sc_gs(t2.reshape(n * 8, 16), sd_a, npad, epad)
    g2 = _combine_gs(a3, disn, p[pre + "_gcn2_b"][None], ga, n)

    return _cnn(g1, g2, p[pre + "_cnn_W"], p[pre + "_cnn_b"][None])


def kernel(het_net, het_x, herb_net, herb_x, target_net, target_x, params):
    p = params
    het_e = _branch(het_x, het_net, p, "het")
    herb_e = _branch(herb_x, herb_net, p, "herb")
    tgt_e = _branch(target_x, target_net, p, "tgt")
    C = het_e[:N_HERB]
    D = het_e[N_HERB:]
    return _final(C, D, herb_e, tgt_e)


# speedup vs baseline: 11.5789x; 1.1220x over previous
import dataclasses
import functools

import jax
import jax.numpy as jnp
from jax import lax
from jax.experimental import pallas as pl
from jax.experimental.pallas import tpu as pltpu
from jax.experimental.pallas import tpu_sc as plsc

H = 128
N_HERB = 2048
NC = 2
NS = 16
NT = NC * NS
LANES = 16


def _mesh():
    return plsc.VectorSubcoreMesh(core_axis_name="c", subcore_axis_name="s")


def _sc_params():
    cp = pltpu.CompilerParams()
    fields = pltpu.CompilerParams.__dataclass_fields__
    if "needs_layout_passes" in fields:
        cp = dataclasses.replace(cp, needs_layout_passes=False)
    if "use_tc_tiling_on_sc" in fields:
        cp = dataclasses.replace(cp, use_tc_tiling_on_sc=False)
    return cp


def _zero_rows(buf):
    R, W = buf.shape
    z = jnp.zeros((LANES,), jnp.float32)

    @pl.loop(0, R)
    def _(i):
        buf[i, :] = z


def _sc_hist(dst, npad):
    (E,) = dst.shape
    CH = 256
    cpt = E // (NT * CH)

    @functools.partial(
        pl.kernel,
        out_type=jax.ShapeDtypeStruct((npad, NT * LANES), jnp.float32),
        mesh=_mesh(),
        compiler_params=_sc_params(),
        scratch_types=[
            pltpu.VMEM((CH,), jnp.int32),
            pltpu.VMEM((npad, LANES), jnp.float32),
        ],
    )
    def k(dst_hbm, out_hbm, d_v, acc_v):
        cid = lax.axis_index("c")
        sid = lax.axis_index("s")
        t = sid * NC + cid
        _zero_rows(acc_v)
        iota = lax.iota(jnp.int32, LANES)
        ones = jnp.full((LANES,), 1.0, jnp.float32)

        @pl.loop(0, cpt)
        def _(j):
            base = (t * cpt + j) * CH
            pltpu.sync_copy(dst_hbm.at[pl.ds(base, CH)], d_v)

            @pl.loop(0, CH)
            def _(e):
                dsp = plsc.load_gather(d_v, [jnp.full((LANES,), e,
                                                      jnp.int32)])
                plsc.addupdate_scatter(acc_v, [dsp, iota], ones)

        pltpu.sync_copy(acc_v, out_hbm.at[:, pl.ds(t * LANES, LANES)])

    return k(dst)


def _sc_gs(tab8, sd, npad, E):
    CH = 256
    NG = 4
    cpt = E // (NG * CH)
    half = cpt // 2

    @functools.partial(
        pl.kernel,
        out_type=jax.ShapeDtypeStruct((NG, npad, H), jnp.float32),
        mesh=_mesh(),
        compiler_params=_sc_params(),
        scratch_types=[
            pltpu.VMEM((2 * CH,), jnp.int32),
            pltpu.VMEM((2 * CH,), jnp.int32),
            pltpu.VMEM((CH,), jnp.int32),
            pltpu.VMEM((CH,), jnp.int32),
            pltpu.VMEM((CH,), jnp.int32),
            pltpu.VMEM((CH,), jnp.int32),
            pltpu.VMEM((CH, LANES), jnp.float32),
            pltpu.VMEM((CH, LANES), jnp.float32),
            pltpu.VMEM((npad, LANES), jnp.float32),
            pltpu.SemaphoreType.DMA,
            pltpu.SemaphoreType.DMA,
            pltpu.SemaphoreType.DMA,
            pltpu.SemaphoreType.DMA,
        ],
    )
    def k(tab_hbm, sd_hbm, out_hbm, sd0, sd1, gi0, gi1, dv0, dv1, rw0, rw1,
          acc_v, sm0, sm1, is0, is1):
        cid = lax.axis_index("c")
        sid = lax.axis_index("s")
        t = sid * NC + cid
        g = t // 8
        cs = t - g * 8
        _zero_rows(acc_v)
        iota = lax.iota(jnp.int32, LANES)
        sdv = (sd0, sd1)
        giv = (gi0, gi1)
        dvv = (dv0, dv1)
        rwv = (rw0, rw1)
        smv = (sm0, sm1)
        ism = (is0, is1)

        def start_idx(b, j):
            base = (g * cpt + j) * CH
            pltpu.make_async_copy(sd_hbm.at[pl.ds(base * 2, 2 * CH)],
                                  sdv[b], ism[b]).start()

        def arm(b):
            pltpu.make_async_copy(sd_hbm.at[pl.ds(0, 2 * CH)], sdv[b],
                                  ism[b]).wait()

            @pl.loop(0, CH, step=LANES)
            def _(o):
                giv[b][pl.ds(o, LANES)] = sdv[b][pl.ds(o, LANES)] * 8 + cs
                dvv[b][pl.ds(o, LANES)] = sdv[b][pl.ds(CH + o, LANES)]

            pltpu.make_async_copy(tab_hbm.at[giv[b]], rwv[b],
                                  smv[b]).start()

        def fin(b):
            pltpu.make_async_copy(tab_hbm.at[giv[b]], rwv[b],
                                  smv[b]).wait()

            @pl.loop(0, CH)
            def _(e):
                dsp = plsc.load_gather(
                    dvv[b], [jnp.full((LANES,), e, jnp.int32)])
                plsc.addupdate_scatter(acc_v, [dsp, iota], rwv[b][e, :])

        start_idx(0, 0)
        start_idx(1, 1)
        arm(0)

        @pl.loop(0, half)
        def _(j2):
            j = j2 * 2
            arm(1)
            start_idx(0, jnp.minimum(j + 2, cpt - 1))
            fin(0)
            arm(0)
            start_idx(1, jnp.minimum(j + 3, cpt - 1))
            fin(1)

        pltpu.make_async_copy(tab_hbm.at[giv[0]], rwv[0], smv[0]).wait()
        pltpu.make_async_copy(sd_hbm.at[pl.ds(0, 2 * CH)], sdv[1],
                              ism[1]).wait()
        pltpu.sync_copy(acc_v, out_hbm.at[g, :, pl.ds(cs * LANES, LANES)])

    return k(tab8, sd)


def _sc_den(sd, als_t, ald_t, smax16, npad, E):
    CH = 128
    G = CH // LANES
    NG = 8
    cpt = E // (NG * CH)

    @functools.partial(
        pl.kernel,
        out_type=(jax.ShapeDtypeStruct((4, E), jnp.float32),
                  jax.ShapeDtypeStruct((npad, NT * LANES), jnp.float32)),
        mesh=_mesh(),
        compiler_params=_sc_params(),
        scratch_types=[
            pltpu.VMEM((2 * CH,), jnp.int32),
            pltpu.VMEM((npad,), jnp.float32),
            pltpu.VMEM((npad,), jnp.float32),
            pltpu.VMEM((LANES,), jnp.float32),
            pltpu.VMEM((CH,), jnp.float32),
            pltpu.VMEM((npad, LANES), jnp.float32),
        ],
    )
    def k(sd_hbm, als_hbm, ald_hbm, sm_hbm, q_hbm, z_hbm,
          sd_v, als_v, ald_v, sm_v, qb, acc_v):
        cid = lax.axis_index("c")
        sid = lax.axis_index("s")
        t = sid * NC + cid
        g = t // 4
        h = t - g * 4
        pltpu.sync_copy(als_hbm.at[h], als_v)
        pltpu.sync_copy(ald_hbm.at[h], ald_v)
        pltpu.sync_copy(sm_hbm, sm_v)
        _zero_rows(acc_v)
        iota = lax.iota(jnp.int32, LANES)
        smsp = plsc.load_gather(sm_v, [jnp.full((LANES,), h, jnp.int32)])

        @pl.loop(0, cpt)
        def _(j):
            base = (g * cpt + j) * CH
            pltpu.sync_copy(sd_hbm.at[pl.ds(base * 2, 2 * CH)], sd_v)

            @pl.loop(0, G)
            def _(gg):
                sl = pl.ds(gg * LANES, LANES)
                av = plsc.load_gather(als_v, [sd_v[sl]])
                dv = plsc.load_gather(
                    ald_v, [sd_v[pl.ds(CH + gg * LANES, LANES)]])
                al = av + dv
                al = jnp.maximum(al, al * 0.2)
                cc = dv + smsp
                cc = jnp.maximum(cc, cc * 0.2)
                qb[sl] = jnp.exp(al - cc)

            @pl.loop(0, CH)
            def _(e):
                ee = jnp.full((LANES,), e, jnp.int32)
                dsp = plsc.load_gather(sd_v, [ee + CH])
                qsp = plsc.load_gather(qb, [ee])
                plsc.addupdate_scatter(acc_v, [dsp, iota], qsp)

            pltpu.sync_copy(qb, q_hbm.at[h, pl.ds(base, CH)])

        pltpu.sync_copy(acc_v, z_hbm.at[:, pl.ds(t * LANES, LANES)])

    return k(sd, als_t, ald_t, smax16)


def _sc_att(q, dst, inv, npad):
    four, E = q.shape
    CH = 128
    G = CH // LANES
    cpt = E // (NT * CH)

    @functools.partial(
        pl.kernel,
        out_type=jax.ShapeDtypeStruct((4, E), jnp.float32),
        mesh=_mesh(),
        compiler_params=_sc_params(),
        scratch_types=[
            pltpu.VMEM((CH,), jnp.int32),
            pltpu.VMEM((4, CH), jnp.float32),
            pltpu.VMEM((npad * 4,), jnp.float32),
            pltpu.SemaphoreType.DMA,
        ],
    )
    def k(q_hbm, dst_hbm, inv_hbm, w_hbm, d_v, q_v, inv_v, sem):
        cid = lax.axis_index("c")
        sid = lax.axis_index("s")
        t = sid * NC + cid
        pltpu.sync_copy(inv_hbm, inv_v)

        @pl.loop(0, cpt)
        def _(j):
            base = (t * cpt + j) * CH
            pltpu.sync_copy(dst_hbm.at[pl.ds(base, CH)], d_v)
            for h in range(4):
                pltpu.make_async_copy(q_hbm.at[h, pl.ds(base, CH)],
                                      q_v.at[h], sem).start()
            for h in range(4):
                pltpu.make_async_copy(q_hbm.at[h, pl.ds(base, CH)],
                                      q_v.at[h], sem).wait()
            for h in range(4):
                @pl.loop(0, G)
                def _(gg):
                    sl = pl.ds(gg * LANES, LANES)
                    iv = plsc.load_gather(inv_v, [d_v[sl] * 4 + h])
                    q_v[h, sl] = q_v[h, sl] * iv

            for h in range(4):
                pltpu.sync_copy(q_v.at[h], w_hbm.at[h, pl.ds(base, CH)])

    return k(q, dst, inv)


def _sc_gsw(tab32, sd, w4, npad, E):
    CH = 128
    NG = 4
    cpt = E // (NG * CH)
    half = cpt // 2

    @functools.partial(
        pl.kernel,
        out_type=jax.ShapeDtypeStruct((NG, npad, H), jnp.float32),
        mesh=_mesh(),
        compiler_params=_sc_params(),
        scratch_types=[
            pltpu.VMEM((2 * CH,), jnp.int32),
            pltpu.VMEM((2 * CH,), jnp.int32),
            pltpu.VMEM((4, CH), jnp.int32),
            pltpu.VMEM((4, CH), jnp.int32),
            pltpu.VMEM((CH,), jnp.int32),
            pltpu.VMEM((CH,), jnp.int32),
            pltpu.VMEM((4 * CH,), jnp.float32),
            pltpu.VMEM((4 * CH,), jnp.float32),
            pltpu.VMEM((4, CH, LANES), jnp.float32),
            pltpu.VMEM((4, CH, LANES), jnp.float32),
            pltpu.VMEM((npad, LANES), jnp.float32),
            pltpu.SemaphoreType.DMA,
            pltpu.SemaphoreType.DMA,
            pltpu.SemaphoreType.DMA,
            pltpu.SemaphoreType.DMA,
        ],
    )
    def k(tab_hbm, sd_hbm, w_hbm, out_hbm, sd0, sd1, gi0, gi1, dv0, dv1,
          wb0, wb1, rw0, rw1, acc_v, sm0, sm1, is0, is1):
        cid = lax.axis_index("c")
        sid = lax.axis_index("s")
        t = sid * NC + cid
        g = t // 8
        cs = t - g * 8
        _zero_rows(acc_v)
        iota = lax.iota(jnp.int32, LANES)
        sdv = (sd0, sd1)
        giv = (gi0, gi1)
        dvv = (dv0, dv1)
        wbv = (wb0, wb1)
        rwv = (rw0, rw1)
        smv = (sm0, sm1)
        ism = (is0, is1)

        def start_idx(b, j):
            base = (g * cpt + j) * CH
            pltpu.make_async_copy(sd_hbm.at[pl.ds(base * 2, 2 * CH)],
                                  sdv[b], ism[b]).start()

        def arm(b, j):
            base = (g * cpt + j) * CH
            pltpu.make_async_copy(sd_hbm.at[pl.ds(0, 2 * CH)], sdv[b],
                                  ism[b]).wait()
            for h in range(4):
                pltpu.make_async_copy(w_hbm.at[h, pl.ds(base, CH)],
                                      wbv[b].at[pl.ds(h * CH, CH)],
                                      smv[b]).start()

                @pl.loop(0, CH, step=LANES)
                def _(o):
                    giv[b][h, pl.ds(o, LANES)] = (
                        sdv[b][pl.ds(o, LANES)] * 32 + (h * 8) + cs)

                pltpu.make_async_copy(tab_hbm.at[giv[b].at[h]],
                                     rwv[b].at[h], smv[b]).start()

            @pl.loop(0, CH, step=LANES)
            def _(o):
                dvv[b][pl.ds(o, LANES)] = sdv[b][pl.ds(CH + o, LANES)]

        def wait_data(b):
            for h in range(4):
                pltpu.make_async_copy(tab_hbm.at[giv[b].at[h]],
                                      rwv[b].at[h], smv[b]).wait()
                pltpu.make_async_copy(w_hbm.at[h, pl.ds(0, CH)],
                                      wbv[b].at[pl.ds(h * CH, CH)],
                                      smv[b]).wait()

        def fin(b):
            wait_data(b)

            @pl.loop(0, CH)
            def _(e):
                ee = jnp.full((LANES,), e, jnp.int32)
                dsp = plsc.load_gather(dvv[b], [ee])
                w0 = plsc.load_gather(wbv[b], [ee])
                acc = rwv[b][0, e, :] * w0
                for h in range(1, 4):
                    wh = plsc.load_gather(wbv[b], [h * CH + ee])
                    acc = acc + rwv[b][h, e, :] * wh
                plsc.addupdate_scatter(acc_v, [dsp, iota], acc)

        start_idx(0, 0)
        start_idx(1, 1)
        arm(0, 0)

        @pl.loop(0, half)
        def _(j2):
            j = j2 * 2
            arm(1, j + 1)
            start_idx(0, jnp.minimum(j + 2, cpt - 1))
            fin(0)
            arm(0, jnp.minimum(j + 2, cpt - 1))
            start_idx(1, jnp.minimum(j + 3, cpt - 1))
            fin(1)

        wait_data(0)
        pltpu.make_async_copy(sd_hbm.at[pl.ds(0, 2 * CH)], sdv[1],
                              ism[1]).wait()
        pltpu.sync_copy(acc_v, out_hbm.at[g, :, pl.ds(cs * LANES, LANES)])

    return k(tab32, sd, w4)


def _matmul_bias(x, w, b, bm=256, bk=512):
    M, K = x.shape
    _, Ho = w.shape
    bk = min(bk, K)
    grid = (M // bm, K // bk)

    def body(x_ref, w_ref, b_ref, o_ref):
        kk = pl.program_id(1)

        @pl.when(kk == 0)
        def _():
            o_ref[...] = jnp.broadcast_to(b_ref[...], o_ref.shape)

        o_ref[...] += jnp.dot(x_ref[...], w_ref[...],
                              preferred_element_type=jnp.float32)

    return pl.pallas_call(
        body,
        grid=grid,
        in_specs=[pl.BlockSpec((bm, bk), lambda i, k: (i, k)),
                  pl.BlockSpec((bk, Ho), lambda i, k: (k, 0)),
                  pl.BlockSpec((1, Ho), lambda i, k: (0, 0))],
        out_specs=pl.BlockSpec((bm, Ho), lambda i, k: (i, 0)),
        out_shape=jax.ShapeDtypeStruct((M, Ho), jnp.float32),
    )(x, w, b)


def _scale_matmul(x, w, s, bm=512):
    M, K = x.shape
    _, Ho = w.shape

    def body(x_ref, w_ref, s_ref, o_ref):
        o_ref[...] = jnp.dot(x_ref[...], w_ref[...],
                             preferred_element_type=jnp.float32) * s_ref[...]

    return pl.pallas_call(
        body,
        grid=(M // bm,),
        in_specs=[pl.BlockSpec((bm, K), lambda i: (i, 0)),
                  pl.BlockSpec((K, Ho), lambda i: (0, 0)),
                  pl.BlockSpec((bm, 1), lambda i: (i, 0))],
        out_specs=pl.BlockSpec((bm, Ho), lambda i: (i, 0)),
        out_shape=jax.ShapeDtypeStruct((M, Ho), jnp.float32),
    )(x, w, s)


def _gat_lin(g1, wg, a_s, a_d, bm=512):
    M, K = g1.shape
    _, W4 = wg.shape

    def body(g_ref, w_ref, as_ref, ad_ref, xw_ref, als_ref, ald_ref,
             sm_ref):
        i = pl.program_id(0)
        xw = jnp.dot(g_ref[...], w_ref[...],
                     preferred_element_type=jnp.float32)
        xw_ref[...] = xw
        als_cols = []
        ald_cols = []
        for h in range(4):
            sl = xw[:, h * H:(h + 1) * H]
            als_cols.append(jnp.sum(sl * as_ref[h:h + 1, :], axis=1,
                                    keepdims=True))
            ald_cols.append(jnp.sum(sl * ad_ref[h:h + 1, :], axis=1,
                                    keepdims=True))
        als_blk = jnp.concatenate(als_cols, axis=1)
        ald_blk = jnp.concatenate(ald_cols, axis=1)
        als_ref[...] = als_blk
        ald_ref[...] = ald_blk
        loc = jnp.max(als_blk, axis=0, keepdims=True)

        @pl.when(i == 0)
        def _():
            sm_ref[...] = loc

        @pl.when(i > 0)
        def _():
            sm_ref[...] = jnp.maximum(sm_ref[...], loc)

    return pl.pallas_call(
        body,
        grid=(M // bm,),
        in_specs=[pl.BlockSpec((bm, K), lambda i: (i, 0)),
                  pl.BlockSpec((K, W4), lambda i: (0, 0)),
                  pl.BlockSpec((4, H), lambda i: (0, 0)),
                  pl.BlockSpec((4, H), lambda i: (0, 0))],
        out_specs=[pl.BlockSpec((bm, W4), lambda i: (i, 0)),
                   pl.BlockSpec((bm, 4), lambda i: (i, 0)),
                   pl.BlockSpec((bm, 4), lambda i: (i, 0)),
                   pl.BlockSpec((1, 4), lambda i: (0, 0))],
        out_shape=[jax.ShapeDtypeStruct((M, W4), jnp.float32),
                   jax.ShapeDtypeStruct((M, 4), jnp.float32),
                   jax.ShapeDtypeStruct((M, 4), jnp.float32),
                   jax.ShapeDtypeStruct((1, 4), jnp.float32)],
    )(g1, wg, a_s, a_d)


def _dis_kernel(degs):
    npad = degs.shape[0]

    def body(p_ref, o_ref):
        deg = jnp.sum(p_ref[...], axis=1, keepdims=True) * (1.0 / LANES)
        o_ref[...] = jnp.where(deg > 0, lax.rsqrt(deg), 0.0)

    return pl.pallas_call(
        body,
        out_shape=jax.ShapeDtypeStruct((npad, 1), jnp.float32),
    )(degs)


def _postden(zout, sel):
    npad = zout.shape[0]

    def body(p_ref, s_ref, o_ref):
        z = jnp.dot(p_ref[...], s_ref[...],
                    preferred_element_type=jnp.float32)
        o_ref[...] = 0.25 / (z + 1e-16)

    return pl.pallas_call(
        body,
        out_shape=jax.ShapeDtypeStruct((npad, 4), jnp.float32),
    )(zout, sel)


def _combine_gs(parts, dis, b, res, n):
    def body(p_ref, d_ref, b_ref, r_ref, o_ref):
        x = jnp.sum(p_ref[...], axis=0)[:n]
        o_ref[...] = jnp.maximum(x * d_ref[...] + b_ref[...] + r_ref[...],
                                 0.0)

    return pl.pallas_call(
        body,
        out_shape=jax.ShapeDtypeStruct((n, H), jnp.float32),
    )(parts, dis, b, res)


def _combine_gat(parts, b, res, n):
    def body(p_ref, b_ref, r_ref, o_ref):
        x = jnp.sum(p_ref[...], axis=0)[:n]
        o_ref[...] = jnp.maximum(x + b_ref[...] + r_ref[...], 0.0)

    return pl.pallas_call(
        body,
        out_shape=jax.ShapeDtypeStruct((n, H), jnp.float32),
    )(parts, b, res)


def _cnn(g1, g2, wc, b):
    M = g1.shape[0]
    dn = (((1,), (1,)), ((), ()))

    def body(g1_ref, g2_ref, w_ref, b_ref, o_ref):
        w0 = w_ref[:, 0, :]
        w1 = w_ref[:, 1, :]
        o_ref[...] = (lax.dot_general(g1_ref[...], w0, dn,
                                      preferred_element_type=jnp.float32)
                      + lax.dot_general(g2_ref[...], w1, dn,
                                        preferred_element_type=jnp.float32)
                      + b_ref[...])

    return pl.pallas_call(
        body,
        out_shape=jax.ShapeDtypeStruct((M, H), jnp.float32),
    )(g1, g2, wc, b)


def _final(C, D, he, te, bm=256, bn=512):
    M = C.shape[0]
    N2 = D.shape[0]
    dn = (((1,), (1,)), ((), ()))

    def body(c_ref, d_ref, h_ref, t_ref, o_ref):
        o_ref[...] = (lax.dot_general(c_ref[...], d_ref[...], dn,
                                      preferred_element_type=jnp.float32)
                      + lax.dot_general(h_ref[...], t_ref[...], dn,
                                        preferred_element_type=jnp.float32))

    return pl.pallas_call(
        body,
        grid=(M // bm, N2 // bn),
        in_specs=[pl.BlockSpec((bm, H), lambda i, j: (i, 0)),
                  pl.BlockSpec((bn, H), lambda i, j: (j, 0)),
                  pl.BlockSpec((bm, H), lambda i, j: (i, 0)),
                  pl.BlockSpec((bn, H), lambda i, j: (j, 0))],
        out_specs=pl.BlockSpec((bm, bn), lambda i, j: (i, j)),
        out_shape=jax.ShapeDtypeStruct((M, N2), jnp.float32),
    )(C, D, he, te)


def _pack_sd(src, dst, ch):
    return jnp.concatenate(
        [src.reshape(-1, 1, ch), dst.reshape(-1, 1, ch)],
        axis=1).reshape(-1)


def _branch(x, ei, p, pre):
    n = x.shape[0]
    npad = n + 128
    e = ei.shape[1]
    ep = e + n
    epad = -(-ep // 8192) * 8192
    loop = jnp.arange(n, dtype=jnp.int32)
    src = jnp.concatenate([ei[0].astype(jnp.int32), loop,
                           jnp.zeros((epad - ep,), jnp.int32)])
    dst = jnp.concatenate([ei[1].astype(jnp.int32), loop,
                           jnp.full((epad - ep,), n, jnp.int32)])
    sd_a = _pack_sd(src, dst, 256)
    sd_b = _pack_sd(src, dst, 128)

    degs = _sc_hist(dst, npad)
    dis = _dis_kernel(degs)
    disn = dis[:n]

    h0 = _matmul_bias(x, p[pre + "_proj_W"], p[pre + "_proj_b"][None])

    t1 = _scale_matmul(h0, p[pre + "_gcn1_W"], disn)
    a1 = _sc_gs(t1.reshape(n * 8, 16), sd_a, npad, epad)
    g1 = _combine_gs(a1, disn, p[pre + "_gcn1_b"][None], h0, n)

    xw4, als, ald, smax = _gat_lin(g1, p[pre + "_gat_W"],
                                   p[pre + "_gat_as"], p[pre + "_gat_ad"])
    rowpad = jnp.zeros((4, npad - n), jnp.float32)
    als_t = jnp.concatenate([als.T, rowpad], axis=1)
    ald_t = jnp.concatenate([ald.T, rowpad], axis=1)
    smax16 = jnp.pad(smax.reshape(-1), (0, LANES - 4))
    q, zout = _sc_den(sd_b, als_t, ald_t, smax16, npad, epad)
    sel = ((jnp.arange(NT * LANES)[:, None] // LANES) % 4
           == jnp.arange(4)[None, :]).astype(jnp.float32) / LANES
    inv = _postden(zout, sel)
    w4 = _sc_att(q, dst, inv.reshape(-1), npad)
    a2 = _sc_gsw(xw4.reshape(n * 32, 16), sd_b, w4, npad, epad)
    ga = _combine_gat(a2, p[pre + "_gat_b"][None], g1, n)

    t2 = _scale_matmul(ga, p[pre + "_gcn2_W"], disn)
    a3 = _sc_gs(t2.reshape(n * 8, 16), sd_a, npad, epad)
    g2 = _combine_gs(a3, disn, p[pre + "_gcn2_b"][None], ga, n)

    return _cnn(g1, g2, p[pre + "_cnn_W"], p[pre + "_cnn_b"][None])


def kernel(het_net, het_x, herb_net, herb_x, target_net, target_x, params):
    p = params
    het_e = _branch(het_x, het_net, p, "het")
    herb_e = _branch(herb_x, herb_net, p, "herb")
    tgt_e = _branch(target_x, target_net, p, "tgt")
    C = het_e[:N_HERB]
    D = het_e[N_HERB:]
    return _final(C, D, herb_e, tgt_e)


# speedup vs baseline: 15.6153x; 1.3486x over previous
import dataclasses
import functools

import jax
import jax.numpy as jnp
from jax import lax
from jax.experimental import pallas as pl
from jax.experimental.pallas import tpu as pltpu
from jax.experimental.pallas import tpu_sc as plsc

H = 128
N_HERB = 2048
NC = 2
NS = 16
NT = NC * NS
LANES = 16


def _mesh():
    return plsc.VectorSubcoreMesh(core_axis_name="c", subcore_axis_name="s")


def _sc_params():
    cp = pltpu.CompilerParams()
    fields = pltpu.CompilerParams.__dataclass_fields__
    if "needs_layout_passes" in fields:
        cp = dataclasses.replace(cp, needs_layout_passes=False)
    if "use_tc_tiling_on_sc" in fields:
        cp = dataclasses.replace(cp, use_tc_tiling_on_sc=False)
    return cp


def _zero_rows(buf):
    R, W = buf.shape
    z = jnp.zeros((LANES,), jnp.float32)

    @pl.loop(0, R)
    def _(i):
        buf[i, :] = z


def _sc_hist(dst, npad):
    (E,) = dst.shape
    CH = 256
    cpt = E // (NT * CH)

    @functools.partial(
        pl.kernel,
        out_type=jax.ShapeDtypeStruct((npad, NT * LANES), jnp.float32),
        mesh=_mesh(),
        compiler_params=_sc_params(),
        scratch_types=[
            pltpu.VMEM((CH,), jnp.int32),
            pltpu.VMEM((npad, LANES), jnp.float32),
        ],
    )
    def k(dst_hbm, out_hbm, d_v, acc_v):
        cid = lax.axis_index("c")
        sid = lax.axis_index("s")
        t = sid * NC + cid
        _zero_rows(acc_v)
        iota = lax.iota(jnp.int32, LANES)
        ones = jnp.full((LANES,), 1.0, jnp.float32)

        @pl.loop(0, cpt)
        def _(j):
            base = (t * cpt + j) * CH
            pltpu.sync_copy(dst_hbm.at[pl.ds(base, CH)], d_v)

            @plsc.parallel_loop(0, CH, unroll=8)
            def _(e):
                dsp = plsc.load_gather(d_v, [jnp.full((LANES,), e,
                                                      jnp.int32)])
                plsc.addupdate_scatter(acc_v, [dsp, iota], ones)

        pltpu.sync_copy(acc_v, out_hbm.at[:, pl.ds(t * LANES, LANES)])

    return k(dst)


def _sc_gs(tab8, sd, npad, E):
    CH = 256
    NG = 4
    cpt = E // (NG * CH)
    half = cpt // 2

    @functools.partial(
        pl.kernel,
        out_type=jax.ShapeDtypeStruct((NG, npad, H), jnp.float32),
        mesh=_mesh(),
        compiler_params=_sc_params(),
        scratch_types=[
            pltpu.VMEM((2 * CH,), jnp.int32),
            pltpu.VMEM((2 * CH,), jnp.int32),
            pltpu.VMEM((CH,), jnp.int32),
            pltpu.VMEM((CH,), jnp.int32),
            pltpu.VMEM((CH,), jnp.int32),
            pltpu.VMEM((CH,), jnp.int32),
            pltpu.VMEM((CH, LANES), jnp.float32),
            pltpu.VMEM((CH, LANES), jnp.float32),
            pltpu.VMEM((npad, LANES), jnp.float32),
            pltpu.SemaphoreType.DMA,
            pltpu.SemaphoreType.DMA,
            pltpu.SemaphoreType.DMA,
            pltpu.SemaphoreType.DMA,
        ],
    )
    def k(tab_hbm, sd_hbm, out_hbm, sd0, sd1, gi0, gi1, dv0, dv1, rw0, rw1,
          acc_v, sm0, sm1, is0, is1):
        cid = lax.axis_index("c")
        sid = lax.axis_index("s")
        t = sid * NC + cid
        g = t // 8
        cs = t - g * 8
        _zero_rows(acc_v)
        iota = lax.iota(jnp.int32, LANES)
        sdv = (sd0, sd1)
        giv = (gi0, gi1)
        dvv = (dv0, dv1)
        rwv = (rw0, rw1)
        smv = (sm0, sm1)
        ism = (is0, is1)

        def start_idx(b, j):
            base = (g * cpt + j) * CH
            pltpu.make_async_copy(sd_hbm.at[pl.ds(base * 2, 2 * CH)],
                                  sdv[b], ism[b]).start()

        def arm(b):
            pltpu.make_async_copy(sd_hbm.at[pl.ds(0, 2 * CH)], sdv[b],
                                  ism[b]).wait()

            @pl.loop(0, CH, step=LANES)
            def _(o):
                giv[b][pl.ds(o, LANES)] = sdv[b][pl.ds(o, LANES)] * 8 + cs
                dvv[b][pl.ds(o, LANES)] = sdv[b][pl.ds(CH + o, LANES)]

            pltpu.make_async_copy(tab_hbm.at[giv[b]], rwv[b],
                                  smv[b]).start()

        def fin(b):
            pltpu.make_async_copy(tab_hbm.at[giv[b]], rwv[b],
                                  smv[b]).wait()

            @plsc.parallel_loop(0, CH, unroll=8)
            def _(e):
                dsp = plsc.load_gather(
                    dvv[b], [jnp.full((LANES,), e, jnp.int32)])
                plsc.addupdate_scatter(acc_v, [dsp, iota], rwv[b][e, :])

        start_idx(0, 0)
        start_idx(1, 1)
        arm(0)

        @pl.loop(0, half)
        def _(j2):
            j = j2 * 2
            arm(1)
            start_idx(0, jnp.minimum(j + 2, cpt - 1))
            fin(0)
            arm(0)
            start_idx(1, jnp.minimum(j + 3, cpt - 1))
            fin(1)

        pltpu.make_async_copy(tab_hbm.at[giv[0]], rwv[0], smv[0]).wait()
        pltpu.make_async_copy(sd_hbm.at[pl.ds(0, 2 * CH)], sdv[1],
                              ism[1]).wait()
        pltpu.sync_copy(acc_v, out_hbm.at[g, :, pl.ds(cs * LANES, LANES)])

    return k(tab8, sd)


def _sc_den(sd, als_t, ald_t, smax16, npad, E):
    CH = 128
    G = CH // LANES
    NG = 8
    cpt = E // (NG * CH)

    @functools.partial(
        pl.kernel,
        out_type=(jax.ShapeDtypeStruct((4, E), jnp.float32),
                  jax.ShapeDtypeStruct((npad, NT * LANES), jnp.float32)),
        mesh=_mesh(),
        compiler_params=_sc_params(),
        scratch_types=[
            pltpu.VMEM((2 * CH,), jnp.int32),
            pltpu.VMEM((npad,), jnp.float32),
            pltpu.VMEM((npad,), jnp.float32),
            pltpu.VMEM((LANES,), jnp.float32),
            pltpu.VMEM((CH,), jnp.float32),
            pltpu.VMEM((npad, LANES), jnp.float32),
        ],
    )
    def k(sd_hbm, als_hbm, ald_hbm, sm_hbm, q_hbm, z_hbm,
          sd_v, als_v, ald_v, sm_v, qb, acc_v):
        cid = lax.axis_index("c")
        sid = lax.axis_index("s")
        t = sid * NC + cid
        g = t // 4
        h = t - g * 4
        pltpu.sync_copy(als_hbm.at[h], als_v)
        pltpu.sync_copy(ald_hbm.at[h], ald_v)
        pltpu.sync_copy(sm_hbm, sm_v)
        _zero_rows(acc_v)
        iota = lax.iota(jnp.int32, LANES)
        smsp = plsc.load_gather(sm_v, [jnp.full((LANES,), h, jnp.int32)])

        @pl.loop(0, cpt)
        def _(j):
            base = (g * cpt + j) * CH
            pltpu.sync_copy(sd_hbm.at[pl.ds(base * 2, 2 * CH)], sd_v)

            @pl.loop(0, G)
            def _(gg):
                sl = pl.ds(gg * LANES, LANES)
                av = plsc.load_gather(als_v, [sd_v[sl]])
                dv = plsc.load_gather(
                    ald_v, [sd_v[pl.ds(CH + gg * LANES, LANES)]])
                al = av + dv
                al = jnp.maximum(al, al * 0.2)
                cc = dv + smsp
                cc = jnp.maximum(cc, cc * 0.2)
                qb[sl] = jnp.exp(al - cc)

            @plsc.parallel_loop(0, CH, unroll=8)
            def _(e):
                ee = jnp.full((LANES,), e, jnp.int32)
                dsp = plsc.load_gather(sd_v, [ee + CH])
                qsp = plsc.load_gather(qb, [ee])
                plsc.addupdate_scatter(acc_v, [dsp, iota], qsp)

            pltpu.sync_copy(qb, q_hbm.at[h, pl.ds(base, CH)])

        pltpu.sync_copy(acc_v, z_hbm.at[:, pl.ds(t * LANES, LANES)])

    return k(sd, als_t, ald_t, smax16)


def _sc_att(q, dst, inv, npad):
    four, E = q.shape
    CH = 128
    G = CH // LANES
    cpt = E // (NT * CH)

    @functools.partial(
        pl.kernel,
        out_type=jax.ShapeDtypeStruct((4, E), jnp.float32),
        mesh=_mesh(),
        compiler_params=_sc_params(),
        scratch_types=[
            pltpu.VMEM((CH,), jnp.int32),
            pltpu.VMEM((4, CH), jnp.float32),
            pltpu.VMEM((npad * 4,), jnp.float32),
            pltpu.SemaphoreType.DMA,
        ],
    )
    def k(q_hbm, dst_hbm, inv_hbm, w_hbm, d_v, q_v, inv_v, sem):
        cid = lax.axis_index("c")
        sid = lax.axis_index("s")
        t = sid * NC + cid
        pltpu.sync_copy(inv_hbm, inv_v)

        @pl.loop(0, cpt)
        def _(j):
            base = (t * cpt + j) * CH
            pltpu.sync_copy(dst_hbm.at[pl.ds(base, CH)], d_v)
            for h in range(4):
                pltpu.make_async_copy(q_hbm.at[h, pl.ds(base, CH)],
                                      q_v.at[h], sem).start()
            for h in range(4):
                pltpu.make_async_copy(q_hbm.at[h, pl.ds(base, CH)],
                                      q_v.at[h], sem).wait()
            for h in range(4):
                @pl.loop(0, G)
                def _(gg):
                    sl = pl.ds(gg * LANES, LANES)
                    iv = plsc.load_gather(inv_v, [d_v[sl] * 4 + h])
                    q_v[h, sl] = q_v[h, sl] * iv

            for h in range(4):
                pltpu.sync_copy(q_v.at[h], w_hbm.at[h, pl.ds(base, CH)])

    return k(q, dst, inv)


def _sc_gsw(tab32, sd, w4, npad, E):
    CH = 128
    NG = 4
    cpt = E // (NG * CH)
    half = cpt // 2

    @functools.partial(
        pl.kernel,
        out_type=jax.ShapeDtypeStruct((NG, npad, H), jnp.float32),
        mesh=_mesh(),
        compiler_params=_sc_params(),
        scratch_types=[
            pltpu.VMEM((2 * CH,), jnp.int32),
            pltpu.VMEM((2 * CH,), jnp.int32),
            pltpu.VMEM((4, CH), jnp.int32),
            pltpu.VMEM((4, CH), jnp.int32),
            pltpu.VMEM((CH,), jnp.int32),
            pltpu.VMEM((CH,), jnp.int32),
            pltpu.VMEM((4 * CH,), jnp.float32),
            pltpu.VMEM((4 * CH,), jnp.float32),
            pltpu.VMEM((4, CH, LANES), jnp.float32),
            pltpu.VMEM((4, CH, LANES), jnp.float32),
            pltpu.VMEM((npad, LANES), jnp.float32),
            pltpu.SemaphoreType.DMA,
            pltpu.SemaphoreType.DMA,
            pltpu.SemaphoreType.DMA,
            pltpu.SemaphoreType.DMA,
        ],
    )
    def k(tab_hbm, sd_hbm, w_hbm, out_hbm, sd0, sd1, gi0, gi1, dv0, dv1,
          wb0, wb1, rw0, rw1, acc_v, sm0, sm1, is0, is1):
        cid = lax.axis_index("c")
        sid = lax.axis_index("s")
        t = sid * NC + cid
        g = t // 8
        cs = t - g * 8
        _zero_rows(acc_v)
        iota = lax.iota(jnp.int32, LANES)
        sdv = (sd0, sd1)
        giv = (gi0, gi1)
        dvv = (dv0, dv1)
        wbv = (wb0, wb1)
        rwv = (rw0, rw1)
        smv = (sm0, sm1)
        ism = (is0, is1)

        def start_idx(b, j):
            base = (g * cpt + j) * CH
            pltpu.make_async_copy(sd_hbm.at[pl.ds(base * 2, 2 * CH)],
                                  sdv[b], ism[b]).start()

        def arm(b, j):
            base = (g * cpt + j) * CH
            pltpu.make_async_copy(sd_hbm.at[pl.ds(0, 2 * CH)], sdv[b],
                                  ism[b]).wait()
            for h in range(4):
                pltpu.make_async_copy(w_hbm.at[h, pl.ds(base, CH)],
                                      wbv[b].at[pl.ds(h * CH, CH)],
                                      smv[b]).start()

                @pl.loop(0, CH, step=LANES)
                def _(o):
                    giv[b][h, pl.ds(o, LANES)] = (
                        sdv[b][pl.ds(o, LANES)] * 32 + (h * 8) + cs)

                pltpu.make_async_copy(tab_hbm.at[giv[b].at[h]],
                                     rwv[b].at[h], smv[b]).start()

            @pl.loop(0, CH, step=LANES)
            def _(o):
                dvv[b][pl.ds(o, LANES)] = sdv[b][pl.ds(CH + o, LANES)]

        def wait_data(b):
            for h in range(4):
                pltpu.make_async_copy(tab_hbm.at[giv[b].at[h]],
                                      rwv[b].at[h], smv[b]).wait()
                pltpu.make_async_copy(w_hbm.at[h, pl.ds(0, CH)],
                                      wbv[b].at[pl.ds(h * CH, CH)],
                                      smv[b]).wait()

        def fin(b):
            wait_data(b)

            @plsc.parallel_loop(0, CH, unroll=4)
            def _(e):
                ee = jnp.full((LANES,), e, jnp.int32)
                dsp = plsc.load_gather(dvv[b], [ee])
                w0 = plsc.load_gather(wbv[b], [ee])
                acc = rwv[b][0, e, :] * w0
                for h in range(1, 4):
                    wh = plsc.load_gather(wbv[b], [h * CH + ee])
                    acc = acc + rwv[b][h, e, :] * wh
                plsc.addupdate_scatter(acc_v, [dsp, iota], acc)

        start_idx(0, 0)
        start_idx(1, 1)
        arm(0, 0)

        @pl.loop(0, half)
        def _(j2):
            j = j2 * 2
            arm(1, j + 1)
            start_idx(0, jnp.minimum(j + 2, cpt - 1))
            fin(0)
            arm(0, jnp.minimum(j + 2, cpt - 1))
            start_idx(1, jnp.minimum(j + 3, cpt - 1))
            fin(1)

        wait_data(0)
        pltpu.make_async_copy(sd_hbm.at[pl.ds(0, 2 * CH)], sdv[1],
                              ism[1]).wait()
        pltpu.sync_copy(acc_v, out_hbm.at[g, :, pl.ds(cs * LANES, LANES)])

    return k(tab32, sd, w4)


def _matmul_bias(x, w, b, bm=256, bk=512):
    M, K = x.shape
    _, Ho = w.shape
    bk = min(bk, K)
    grid = (M // bm, K // bk)

    def body(x_ref, w_ref, b_ref, o_ref):
        kk = pl.program_id(1)

        @pl.when(kk == 0)
        def _():
            o_ref[...] = jnp.broadcast_to(b_ref[...], o_ref.shape)

        o_ref[...] += jnp.dot(x_ref[...], w_ref[...],
                              preferred_element_type=jnp.float32)

    return pl.pallas_call(
        body,
        grid=grid,
        in_specs=[pl.BlockSpec((bm, bk), lambda i, k: (i, k)),
                  pl.BlockSpec((bk, Ho), lambda i, k: (k, 0)),
                  pl.BlockSpec((1, Ho), lambda i, k: (0, 0))],
        out_specs=pl.BlockSpec((bm, Ho), lambda i, k: (i, 0)),
        out_shape=jax.ShapeDtypeStruct((M, Ho), jnp.float32),
    )(x, w, b)


def _scale_matmul(x, w, s, bm=512):
    M, K = x.shape
    _, Ho = w.shape

    def body(x_ref, w_ref, s_ref, o_ref):
        o_ref[...] = jnp.dot(x_ref[...], w_ref[...],
                             preferred_element_type=jnp.float32) * s_ref[...]

    return pl.pallas_call(
        body,
        grid=(M // bm,),
        in_specs=[pl.BlockSpec((bm, K), lambda i: (i, 0)),
                  pl.BlockSpec((K, Ho), lambda i: (0, 0)),
                  pl.BlockSpec((bm, 1), lambda i: (i, 0))],
        out_specs=pl.BlockSpec((bm, Ho), lambda i: (i, 0)),
        out_shape=jax.ShapeDtypeStruct((M, Ho), jnp.float32),
    )(x, w, s)


def _gat_lin(g1, wg, a_s, a_d, bm=512):
    M, K = g1.shape
    _, W4 = wg.shape

    def body(g_ref, w_ref, as_ref, ad_ref, xw_ref, als_ref, ald_ref,
             sm_ref):
        i = pl.program_id(0)
        xw = jnp.dot(g_ref[...], w_ref[...],
                     preferred_element_type=jnp.float32)
        xw_ref[...] = xw
        als_cols = []
        ald_cols = []
        for h in range(4):
            sl = xw[:, h * H:(h + 1) * H]
            als_cols.append(jnp.sum(sl * as_ref[h:h + 1, :], axis=1,
                                    keepdims=True))
            ald_cols.append(jnp.sum(sl * ad_ref[h:h + 1, :], axis=1,
                                    keepdims=True))
        als_blk = jnp.concatenate(als_cols, axis=1)
        ald_blk = jnp.concatenate(ald_cols, axis=1)
        als_ref[...] = als_blk
        ald_ref[...] = ald_blk
        loc = jnp.max(als_blk, axis=0, keepdims=True)

        @pl.when(i == 0)
        def _():
            sm_ref[...] = loc

        @pl.when(i > 0)
        def _():
            sm_ref[...] = jnp.maximum(sm_ref[...], loc)

    return pl.pallas_call(
        body,
        grid=(M // bm,),
        in_specs=[pl.BlockSpec((bm, K), lambda i: (i, 0)),
                  pl.BlockSpec((K, W4), lambda i: (0, 0)),
                  pl.BlockSpec((4, H), lambda i: (0, 0)),
                  pl.BlockSpec((4, H), lambda i: (0, 0))],
        out_specs=[pl.BlockSpec((bm, W4), lambda i: (i, 0)),
                   pl.BlockSpec((bm, 4), lambda i: (i, 0)),
                   pl.BlockSpec((bm, 4), lambda i: (i, 0)),
                   pl.BlockSpec((1, 4), lambda i: (0, 0))],
        out_shape=[jax.ShapeDtypeStruct((M, W4), jnp.float32),
                   jax.ShapeDtypeStruct((M, 4), jnp.float32),
                   jax.ShapeDtypeStruct((M, 4), jnp.float32),
                   jax.ShapeDtypeStruct((1, 4), jnp.float32)],
    )(g1, wg, a_s, a_d)


def _dis_kernel(degs):
    npad = degs.shape[0]

    def body(p_ref, o_ref):
        deg = jnp.sum(p_ref[...], axis=1, keepdims=True) * (1.0 / LANES)
        o_ref[...] = jnp.where(deg > 0, lax.rsqrt(deg), 0.0)

    return pl.pallas_call(
        body,
        out_shape=jax.ShapeDtypeStruct((npad, 1), jnp.float32),
    )(degs)


def _postden(zout, sel):
    npad = zout.shape[0]

    def body(p_ref, s_ref, o_ref):
        z = jnp.dot(p_ref[...], s_ref[...],
                    preferred_element_type=jnp.float32)
        o_ref[...] = 0.25 / (z + 1e-16)

    return pl.pallas_call(
        body,
        out_shape=jax.ShapeDtypeStruct((npad, 4), jnp.float32),
    )(zout, sel)


def _combine_gs(parts, dis, b, res, n):
    def body(p_ref, d_ref, b_ref, r_ref, o_ref):
        x = jnp.sum(p_ref[...], axis=0)[:n]
        o_ref[...] = jnp.maximum(x * d_ref[...] + b_ref[...] + r_ref[...],
                                 0.0)

    return pl.pallas_call(
        body,
        out_shape=jax.ShapeDtypeStruct((n, H), jnp.float32),
    )(parts, dis, b, res)


def _combine_gat(parts, b, res, n):
    def body(p_ref, b_ref, r_ref, o_ref):
        x = jnp.sum(p_ref[...], axis=0)[:n]
        o_ref[...] = jnp.maximum(x + b_ref[...] + r_ref[...], 0.0)

    return pl.pallas_call(
        body,
        out_shape=jax.ShapeDtypeStruct((n, H), jnp.float32),
    )(parts, b, res)


def _cnn(g1, g2, wc, b):
    M = g1.shape[0]
    dn = (((1,), (1,)), ((), ()))

    def body(g1_ref, g2_ref, w_ref, b_ref, o_ref):
        w0 = w_ref[:, 0, :]
        w1 = w_ref[:, 1, :]
        o_ref[...] = (lax.dot_general(g1_ref[...], w0, dn,
                                      preferred_element_type=jnp.float32)
                      + lax.dot_general(g2_ref[...], w1, dn,
                                        preferred_element_type=jnp.float32)
                      + b_ref[...])

    return pl.pallas_call(
        body,
        out_shape=jax.ShapeDtypeStruct((M, H), jnp.float32),
    )(g1, g2, wc, b)


def _final(C, D, he, te, bm=256, bn=512):
    M = C.shape[0]
    N2 = D.shape[0]
    dn = (((1,), (1,)), ((), ()))

    def body(c_ref, d_ref, h_ref, t_ref, o_ref):
        o_ref[...] = (lax.dot_general(c_ref[...], d_ref[...], dn,
                                      preferred_element_type=jnp.float32)
                      + lax.dot_general(h_ref[...], t_ref[...], dn,
                                        preferred_element_type=jnp.float32))

    return pl.pallas_call(
        body,
        grid=(M // bm, N2 // bn),
        in_specs=[pl.BlockSpec((bm, H), lambda i, j: (i, 0)),
                  pl.BlockSpec((bn, H), lambda i, j: (j, 0)),
                  pl.BlockSpec((bm, H), lambda i, j: (i, 0)),
                  pl.BlockSpec((bn, H), lambda i, j: (j, 0))],
        out_specs=pl.BlockSpec((bm, bn), lambda i, j: (i, j)),
        out_shape=jax.ShapeDtypeStruct((M, N2), jnp.float32),
    )(C, D, he, te)


def _pack_sd(src, dst, ch):
    return jnp.concatenate(
        [src.reshape(-1, 1, ch), dst.reshape(-1, 1, ch)],
        axis=1).reshape(-1)


def _branch(x, ei, p, pre):
    n = x.shape[0]
    npad = n + 128
    e = ei.shape[1]
    ep = e + n
    epad = -(-ep // 8192) * 8192
    loop = jnp.arange(n, dtype=jnp.int32)
    src = jnp.concatenate([ei[0].astype(jnp.int32), loop,
                           jnp.zeros((epad - ep,), jnp.int32)])
    dst = jnp.concatenate([ei[1].astype(jnp.int32), loop,
                           jnp.full((epad - ep,), n, jnp.int32)])
    sd_a = _pack_sd(src, dst, 256)
    sd_b = _pack_sd(src, dst, 128)

    degs = _sc_hist(dst, npad)
    dis = _dis_kernel(degs)
    disn = dis[:n]

    h0 = _matmul_bias(x, p[pre + "_proj_W"], p[pre + "_proj_b"][None])

    t1 = _scale_matmul(h0, p[pre + "_gcn1_W"], disn)
    a1 = _sc_gs(t1.reshape(n * 8, 16), sd_a, npad, epad)
    g1 = _combine_gs(a1, disn, p[pre + "_gcn1_b"][None], h0, n)

    xw4, als, ald, smax = _gat_lin(g1, p[pre + "_gat_W"],
                                   p[pre + "_gat_as"], p[pre + "_gat_ad"])
    rowpad = jnp.zeros((4, npad - n), jnp.float32)
    als_t = jnp.concatenate([als.T, rowpad], axis=1)
    ald_t = jnp.concatenate([ald.T, rowpad], axis=1)
    smax16 = jnp.pad(smax.reshape(-1), (0, LANES - 4))
    q, zout = _sc_den(sd_b, als_t, ald_t, smax16, npad, epad)
    sel = ((jnp.arange(NT * LANES)[:, None] // LANES) % 4
           == jnp.arange(4)[None, :]).astype(jnp.float32) / LANES
    inv = _postden(zout, sel)
    w4 = _sc_att(q, dst, inv.reshape(-1), npad)
    a2 = _sc_gsw(xw4.reshape(n * 32, 16), sd_b, w4, npad, epad)
    ga = _combine_gat(a2, p[pre + "_gat_b"][None], g1, n)

    t2 = _scale_matmul(ga, p[pre + "_gcn2_W"], disn)
    a3 = _sc_gs(t2.reshape(n * 8, 16), sd_a, npad, epad)
    g2 = _combine_gs(a3, disn, p[pre + "_gcn2_b"][None], ga, n)

    return _cnn(g1, g2, p[pre + "_cnn_W"], p[pre + "_cnn_b"][None])


def kernel(het_net, het_x, herb_net, herb_x, target_net, target_x, params):
    p = params
    het_e = _branch(het_x, het_net, p, "het")
    herb_e = _branch(herb_x, herb_net, p, "herb")
    tgt_e = _branch(target_x, target_net, p, "tgt")
    C = het_e[:N_HERB]
    D = het_e[N_HERB:]
    return _final(C, D, herb_e, tgt_e)


# speedup vs baseline: 15.7726x; 1.0101x over previous
import dataclasses
import functools

import jax
import jax.numpy as jnp
from jax import lax
from jax.experimental import pallas as pl
from jax.experimental.pallas import tpu as pltpu
from jax.experimental.pallas import tpu_sc as plsc

H = 128
N_HERB = 2048
NC = 2
NS = 16
NT = NC * NS
LANES = 16


def _mesh():
    return plsc.VectorSubcoreMesh(core_axis_name="c", subcore_axis_name="s")


def _sc_params():
    cp = pltpu.CompilerParams()
    fields = pltpu.CompilerParams.__dataclass_fields__
    if "needs_layout_passes" in fields:
        cp = dataclasses.replace(cp, needs_layout_passes=False)
    if "use_tc_tiling_on_sc" in fields:
        cp = dataclasses.replace(cp, use_tc_tiling_on_sc=False)
    return cp


def _zero_rows(buf):
    R, W = buf.shape
    z = jnp.zeros((LANES,), jnp.float32)

    @pl.loop(0, R)
    def _(i):
        buf[i, :] = z


def _sc_hist(dst, npad):
    (E,) = dst.shape
    CH = 256
    cpt = E // (NT * CH)

    @functools.partial(
        pl.kernel,
        out_type=jax.ShapeDtypeStruct((npad, NT * LANES), jnp.float32),
        mesh=_mesh(),
        compiler_params=_sc_params(),
        scratch_types=[
            pltpu.VMEM((CH,), jnp.int32),
            pltpu.VMEM((npad, LANES), jnp.float32),
        ],
    )
    def k(dst_hbm, out_hbm, d_v, acc_v):
        cid = lax.axis_index("c")
        sid = lax.axis_index("s")
        t = sid * NC + cid
        _zero_rows(acc_v)
        iota = lax.iota(jnp.int32, LANES)
        ones = jnp.full((LANES,), 1.0, jnp.float32)

        @pl.loop(0, cpt)
        def _(j):
            base = (t * cpt + j) * CH
            pltpu.sync_copy(dst_hbm.at[pl.ds(base, CH)], d_v)

            @plsc.parallel_loop(0, CH, unroll=8)
            def _(e):
                dsp = plsc.load_gather(d_v, [jnp.full((LANES,), e,
                                                      jnp.int32)])
                plsc.addupdate_scatter(acc_v, [dsp, iota], ones)

        pltpu.sync_copy(acc_v, out_hbm.at[:, pl.ds(t * LANES, LANES)])

    return k(dst)


def _sc_gs(tab8, sd, npad, E):
    CH = 256
    NG = 4
    cpt = E // (NG * CH)
    half = cpt // 2

    @functools.partial(
        pl.kernel,
        out_type=jax.ShapeDtypeStruct((NG, npad, H), jnp.float32),
        mesh=_mesh(),
        compiler_params=_sc_params(),
        scratch_types=[
            pltpu.VMEM((2 * CH,), jnp.int32),
            pltpu.VMEM((2 * CH,), jnp.int32),
            pltpu.VMEM((CH,), jnp.int32),
            pltpu.VMEM((CH,), jnp.int32),
            pltpu.VMEM((CH,), jnp.int32),
            pltpu.VMEM((CH,), jnp.int32),
            pltpu.VMEM((CH, LANES), jnp.float32),
            pltpu.VMEM((CH, LANES), jnp.float32),
            pltpu.VMEM((npad, LANES), jnp.float32),
            pltpu.SemaphoreType.DMA,
            pltpu.SemaphoreType.DMA,
            pltpu.SemaphoreType.DMA,
            pltpu.SemaphoreType.DMA,
        ],
    )
    def k(tab_hbm, sd_hbm, out_hbm, sd0, sd1, gi0, gi1, dv0, dv1, rw0, rw1,
          acc_v, sm0, sm1, is0, is1):
        cid = lax.axis_index("c")
        sid = lax.axis_index("s")
        t = sid * NC + cid
        g = t // 8
        cs = t - g * 8
        _zero_rows(acc_v)
        iota = lax.iota(jnp.int32, LANES)
        sdv = (sd0, sd1)
        giv = (gi0, gi1)
        dvv = (dv0, dv1)
        rwv = (rw0, rw1)
        smv = (sm0, sm1)
        ism = (is0, is1)

        def start_idx(b, j):
            base = (g * cpt + j) * CH
            pltpu.make_async_copy(sd_hbm.at[pl.ds(base * 2, 2 * CH)],
                                  sdv[b], ism[b]).start()

        def arm(b):
            pltpu.make_async_copy(sd_hbm.at[pl.ds(0, 2 * CH)], sdv[b],
                                  ism[b]).wait()

            @pl.loop(0, CH, step=LANES)
            def _(o):
                giv[b][pl.ds(o, LANES)] = sdv[b][pl.ds(o, LANES)] * 8 + cs
                dvv[b][pl.ds(o, LANES)] = sdv[b][pl.ds(CH + o, LANES)]

            pltpu.make_async_copy(tab_hbm.at[giv[b]], rwv[b],
                                  smv[b]).start()

        def fin(b):
            pltpu.make_async_copy(tab_hbm.at[giv[b]], rwv[b],
                                  smv[b]).wait()

            @plsc.parallel_loop(0, CH, unroll=8)
            def _(e):
                dsp = plsc.load_gather(
                    dvv[b], [jnp.full((LANES,), e, jnp.int32)])
                plsc.addupdate_scatter(acc_v, [dsp, iota], rwv[b][e, :])

        start_idx(0, 0)
        start_idx(1, 1)
        arm(0)

        @pl.loop(0, half)
        def _(j2):
            j = j2 * 2
            arm(1)
            start_idx(0, jnp.minimum(j + 2, cpt - 1))
            fin(0)
            arm(0)
            start_idx(1, jnp.minimum(j + 3, cpt - 1))
            fin(1)

        pltpu.make_async_copy(tab_hbm.at[giv[0]], rwv[0], smv[0]).wait()
        pltpu.make_async_copy(sd_hbm.at[pl.ds(0, 2 * CH)], sdv[1],
                              ism[1]).wait()
        pltpu.sync_copy(acc_v, out_hbm.at[g, :, pl.ds(cs * LANES, LANES)])

    return k(tab8, sd)


def _sc_den(sd, als_t, ald_t, smax16, npad, E):
    CH = 128
    G = CH // LANES
    NG = 8
    cpt = E // (NG * CH)

    @functools.partial(
        pl.kernel,
        out_type=(jax.ShapeDtypeStruct((4, E), jnp.float32),
                  jax.ShapeDtypeStruct((npad, NT * LANES), jnp.float32)),
        mesh=_mesh(),
        compiler_params=_sc_params(),
        scratch_types=[
            pltpu.VMEM((2 * CH,), jnp.int32),
            pltpu.VMEM((npad,), jnp.float32),
            pltpu.VMEM((npad,), jnp.float32),
            pltpu.VMEM((LANES,), jnp.float32),
            pltpu.VMEM((CH,), jnp.float32),
            pltpu.VMEM((npad, LANES), jnp.float32),
        ],
    )
    def k(sd_hbm, als_hbm, ald_hbm, sm_hbm, q_hbm, z_hbm,
          sd_v, als_v, ald_v, sm_v, qb, acc_v):
        cid = lax.axis_index("c")
        sid = lax.axis_index("s")
        t = sid * NC + cid
        g = t // 4
        h = t - g * 4
        pltpu.sync_copy(als_hbm.at[h], als_v)
        pltpu.sync_copy(ald_hbm.at[h], ald_v)
        pltpu.sync_copy(sm_hbm, sm_v)
        _zero_rows(acc_v)
        iota = lax.iota(jnp.int32, LANES)
        smsp = plsc.load_gather(sm_v, [jnp.full((LANES,), h, jnp.int32)])

        @pl.loop(0, cpt)
        def _(j):
            base = (g * cpt + j) * CH
            pltpu.sync_copy(sd_hbm.at[pl.ds(base * 2, 2 * CH)], sd_v)

            @plsc.parallel_loop(0, G, unroll=4)
            def _(gg):
                sl = pl.ds(gg * LANES, LANES)
                av = plsc.load_gather(als_v, [sd_v[sl]])
                dv = plsc.load_gather(
                    ald_v, [sd_v[pl.ds(CH + gg * LANES, LANES)]])
                al = av + dv
                al = jnp.maximum(al, al * 0.2)
                cc = dv + smsp
                cc = jnp.maximum(cc, cc * 0.2)
                qb[sl] = jnp.exp(al - cc)

            @plsc.parallel_loop(0, CH, unroll=8)
            def _(e):
                ee = jnp.full((LANES,), e, jnp.int32)
                dsp = plsc.load_gather(sd_v, [ee + CH])
                qsp = plsc.load_gather(qb, [ee])
                plsc.addupdate_scatter(acc_v, [dsp, iota], qsp)

            pltpu.sync_copy(qb, q_hbm.at[h, pl.ds(base, CH)])

        pltpu.sync_copy(acc_v, z_hbm.at[:, pl.ds(t * LANES, LANES)])

    return k(sd, als_t, ald_t, smax16)


def _sc_att(q, dst, inv, npad):
    four, E = q.shape
    CH = 128
    G = CH // LANES
    cpt = E // (NT * CH)

    @functools.partial(
        pl.kernel,
        out_type=jax.ShapeDtypeStruct((4, E), jnp.float32),
        mesh=_mesh(),
        compiler_params=_sc_params(),
        scratch_types=[
            pltpu.VMEM((CH,), jnp.int32),
            pltpu.VMEM((4, CH), jnp.float32),
            pltpu.VMEM((npad * 4,), jnp.float32),
            pltpu.SemaphoreType.DMA,
        ],
    )
    def k(q_hbm, dst_hbm, inv_hbm, w_hbm, d_v, q_v, inv_v, sem):
        cid = lax.axis_index("c")
        sid = lax.axis_index("s")
        t = sid * NC + cid
        pltpu.sync_copy(inv_hbm, inv_v)

        @pl.loop(0, cpt)
        def _(j):
            base = (t * cpt + j) * CH
            pltpu.sync_copy(dst_hbm.at[pl.ds(base, CH)], d_v)
            for h in range(4):
                pltpu.make_async_copy(q_hbm.at[h, pl.ds(base, CH)],
                                      q_v.at[h], sem).start()
            for h in range(4):
                pltpu.make_async_copy(q_hbm.at[h, pl.ds(base, CH)],
                                      q_v.at[h], sem).wait()
            for h in range(4):
                @plsc.parallel_loop(0, G, unroll=4)
                def _(gg):
                    sl = pl.ds(gg * LANES, LANES)
                    iv = plsc.load_gather(inv_v, [d_v[sl] * 4 + h])
                    q_v[h, sl] = q_v[h, sl] * iv

            for h in range(4):
                pltpu.sync_copy(q_v.at[h], w_hbm.at[h, pl.ds(base, CH)])

    return k(q, dst, inv)


def _sc_gsw(tab32, sd, w4, npad, E):
    CH = 128
    NG = 4
    cpt = E // (NG * CH)
    half = cpt // 2

    @functools.partial(
        pl.kernel,
        out_type=jax.ShapeDtypeStruct((NG, npad, H), jnp.float32),
        mesh=_mesh(),
        compiler_params=_sc_params(),
        scratch_types=[
            pltpu.VMEM((2 * CH,), jnp.int32),
            pltpu.VMEM((2 * CH,), jnp.int32),
            pltpu.VMEM((4, CH), jnp.int32),
            pltpu.VMEM((4, CH), jnp.int32),
            pltpu.VMEM((CH,), jnp.int32),
            pltpu.VMEM((CH,), jnp.int32),
            pltpu.VMEM((4 * CH,), jnp.float32),
            pltpu.VMEM((4 * CH,), jnp.float32),
            pltpu.VMEM((4, CH, LANES), jnp.float32),
            pltpu.VMEM((4, CH, LANES), jnp.float32),
            pltpu.VMEM((npad, LANES), jnp.float32),
            pltpu.SemaphoreType.DMA,
            pltpu.SemaphoreType.DMA,
            pltpu.SemaphoreType.DMA,
            pltpu.SemaphoreType.DMA,
        ],
    )
    def k(tab_hbm, sd_hbm, w_hbm, out_hbm, sd0, sd1, gi0, gi1, dv0, dv1,
          wb0, wb1, rw0, rw1, acc_v, sm0, sm1, is0, is1):
        cid = lax.axis_index("c")
        sid = lax.axis_index("s")
        t = sid * NC + cid
        g = t // 8
        cs = t - g * 8
        _zero_rows(acc_v)
        iota = lax.iota(jnp.int32, LANES)
        sdv = (sd0, sd1)
        giv = (gi0, gi1)
        dvv = (dv0, dv1)
        wbv = (wb0, wb1)
        rwv = (rw0, rw1)
        smv = (sm0, sm1)
        ism = (is0, is1)

        def start_idx(b, j):
            base = (g * cpt + j) * CH
            pltpu.make_async_copy(sd_hbm.at[pl.ds(base * 2, 2 * CH)],
                                  sdv[b], ism[b]).start()

        def arm(b, j):
            base = (g * cpt + j) * CH
            pltpu.make_async_copy(sd_hbm.at[pl.ds(0, 2 * CH)], sdv[b],
                                  ism[b]).wait()
            for h in range(4):
                pltpu.make_async_copy(w_hbm.at[h, pl.ds(base, CH)],
                                      wbv[b].at[pl.ds(h * CH, CH)],
                                      smv[b]).start()

                @pl.loop(0, CH, step=LANES)
                def _(o):
                    giv[b][h, pl.ds(o, LANES)] = (
                        sdv[b][pl.ds(o, LANES)] * 32 + (h * 8) + cs)

                pltpu.make_async_copy(tab_hbm.at[giv[b].at[h]],
                                     rwv[b].at[h], smv[b]).start()

            @pl.loop(0, CH, step=LANES)
            def _(o):
                dvv[b][pl.ds(o, LANES)] = sdv[b][pl.ds(CH + o, LANES)]

        def wait_data(b):
            for h in range(4):
                pltpu.make_async_copy(tab_hbm.at[giv[b].at[h]],
                                      rwv[b].at[h], smv[b]).wait()
                pltpu.make_async_copy(w_hbm.at[h, pl.ds(0, CH)],
                                      wbv[b].at[pl.ds(h * CH, CH)],
                                      smv[b]).wait()

        def fin(b):
            wait_data(b)

            @plsc.parallel_loop(0, CH, unroll=4)
            def _(e):
                ee = jnp.full((LANES,), e, jnp.int32)
                dsp = plsc.load_gather(dvv[b], [ee])
                w0 = plsc.load_gather(wbv[b], [ee])
                acc = rwv[b][0, e, :] * w0
                for h in range(1, 4):
                    wh = plsc.load_gather(wbv[b], [h * CH + ee])
                    acc = acc + rwv[b][h, e, :] * wh
                plsc.addupdate_scatter(acc_v, [dsp, iota], acc)

        start_idx(0, 0)
        start_idx(1, 1)
        arm(0, 0)

        @pl.loop(0, half)
        def _(j2):
            j = j2 * 2
            arm(1, j + 1)
            start_idx(0, jnp.minimum(j + 2, cpt - 1))
            fin(0)
            arm(0, jnp.minimum(j + 2, cpt - 1))
            start_idx(1, jnp.minimum(j + 3, cpt - 1))
            fin(1)

        wait_data(0)
        pltpu.make_async_copy(sd_hbm.at[pl.ds(0, 2 * CH)], sdv[1],
                              ism[1]).wait()
        pltpu.sync_copy(acc_v, out_hbm.at[g, :, pl.ds(cs * LANES, LANES)])

    return k(tab32, sd, w4)


def _matmul_bias(x, w, b, bm=256, bk=512):
    M, K = x.shape
    _, Ho = w.shape
    bk = min(bk, K)
    grid = (M // bm, K // bk)

    def body(x_ref, w_ref, b_ref, o_ref):
        kk = pl.program_id(1)

        @pl.when(kk == 0)
        def _():
            o_ref[...] = jnp.broadcast_to(b_ref[...], o_ref.shape)

        o_ref[...] += jnp.dot(x_ref[...], w_ref[...],
                              preferred_element_type=jnp.float32)

    return pl.pallas_call(
        body,
        grid=grid,
        in_specs=[pl.BlockSpec((bm, bk), lambda i, k: (i, k)),
                  pl.BlockSpec((bk, Ho), lambda i, k: (k, 0)),
                  pl.BlockSpec((1, Ho), lambda i, k: (0, 0))],
        out_specs=pl.BlockSpec((bm, Ho), lambda i, k: (i, 0)),
        out_shape=jax.ShapeDtypeStruct((M, Ho), jnp.float32),
    )(x, w, b)


def _scale_matmul(x, w, s, bm=512):
    M, K = x.shape
    _, Ho = w.shape

    def body(x_ref, w_ref, s_ref, o_ref):
        o_ref[...] = jnp.dot(x_ref[...], w_ref[...],
                             preferred_element_type=jnp.float32) * s_ref[...]

    return pl.pallas_call(
        body,
        grid=(M // bm,),
        in_specs=[pl.BlockSpec((bm, K), lambda i: (i, 0)),
                  pl.BlockSpec((K, Ho), lambda i: (0, 0)),
                  pl.BlockSpec((bm, 1), lambda i: (i, 0))],
        out_specs=pl.BlockSpec((bm, Ho), lambda i: (i, 0)),
        out_shape=jax.ShapeDtypeStruct((M, Ho), jnp.float32),
    )(x, w, s)


def _gat_lin(g1, wg, a_s, a_d, bm=512):
    M, K = g1.shape
    _, W4 = wg.shape

    def body(g_ref, w_ref, as_ref, ad_ref, xw_ref, als_ref, ald_ref,
             sm_ref):
        i = pl.program_id(0)
        xw = jnp.dot(g_ref[...], w_ref[...],
                     preferred_element_type=jnp.float32)
        xw_ref[...] = xw
        als_cols = []
        ald_cols = []
        for h in range(4):
            sl = xw[:, h * H:(h + 1) * H]
            als_cols.append(jnp.sum(sl * as_ref[h:h + 1, :], axis=1,
                                    keepdims=True))
            ald_cols.append(jnp.sum(sl * ad_ref[h:h + 1, :], axis=1,
                                    keepdims=True))
        als_blk = jnp.concatenate(als_cols, axis=1)
        ald_blk = jnp.concatenate(ald_cols, axis=1)
        als_ref[...] = als_blk
        ald_ref[...] = ald_blk
        loc = jnp.max(als_blk, axis=0, keepdims=True)

        @pl.when(i == 0)
        def _():
            sm_ref[...] = loc

        @pl.when(i > 0)
        def _():
            sm_ref[...] = jnp.maximum(sm_ref[...], loc)

    return pl.pallas_call(
        body,
        grid=(M // bm,),
        in_specs=[pl.BlockSpec((bm, K), lambda i: (i, 0)),
                  pl.BlockSpec((K, W4), lambda i: (0, 0)),
                  pl.BlockSpec((4, H), lambda i: (0, 0)),
                  pl.BlockSpec((4, H), lambda i: (0, 0))],
        out_specs=[pl.BlockSpec((bm, W4), lambda i: (i, 0)),
                   pl.BlockSpec((bm, 4), lambda i: (i, 0)),
                   pl.BlockSpec((bm, 4), lambda i: (i, 0)),
                   pl.BlockSpec((1, 4), lambda i: (0, 0))],
        out_shape=[jax.ShapeDtypeStruct((M, W4), jnp.float32),
                   jax.ShapeDtypeStruct((M, 4), jnp.float32),
                   jax.ShapeDtypeStruct((M, 4), jnp.float32),
                   jax.ShapeDtypeStruct((1, 4), jnp.float32)],
    )(g1, wg, a_s, a_d)


def _dis_kernel(degs):
    npad = degs.shape[0]

    def body(p_ref, o_ref):
        deg = jnp.sum(p_ref[...], axis=1, keepdims=True) * (1.0 / LANES)
        o_ref[...] = jnp.where(deg > 0, lax.rsqrt(deg), 0.0)

    return pl.pallas_call(
        body,
        out_shape=jax.ShapeDtypeStruct((npad, 1), jnp.float32),
    )(degs)


def _postden(zout, sel):
    npad = zout.shape[0]

    def body(p_ref, s_ref, o_ref):
        z = jnp.dot(p_ref[...], s_ref[...],
                    preferred_element_type=jnp.float32)
        o_ref[...] = 0.25 / (z + 1e-16)

    return pl.pallas_call(
        body,
        out_shape=jax.ShapeDtypeStruct((npad, 4), jnp.float32),
    )(zout, sel)


def _combine_gs(parts, dis, b, res, n):
    def body(p_ref, d_ref, b_ref, r_ref, o_ref):
        x = jnp.sum(p_ref[...], axis=0)[:n]
        o_ref[...] = jnp.maximum(x * d_ref[...] + b_ref[...] + r_ref[...],
                                 0.0)

    return pl.pallas_call(
        body,
        out_shape=jax.ShapeDtypeStruct((n, H), jnp.float32),
    )(parts, dis, b, res)


def _combine_gat(parts, b, res, n):
    def body(p_ref, b_ref, r_ref, o_ref):
        x = jnp.sum(p_ref[...], axis=0)[:n]
        o_ref[...] = jnp.maximum(x + b_ref[...] + r_ref[...], 0.0)

    return pl.pallas_call(
        body,
        out_shape=jax.ShapeDtypeStruct((n, H), jnp.float32),
    )(parts, b, res)


def _cnn(g1, g2, wc, b):
    M = g1.shape[0]
    dn = (((1,), (1,)), ((), ()))

    def body(g1_ref, g2_ref, w_ref, b_ref, o_ref):
        w0 = w_ref[:, 0, :]
        w1 = w_ref[:, 1, :]
        o_ref[...] = (lax.dot_general(g1_ref[...], w0, dn,
                                      preferred_element_type=jnp.float32)
                      + lax.dot_general(g2_ref[...], w1, dn,
                                        preferred_element_type=jnp.float32)
                      + b_ref[...])

    return pl.pallas_call(
        body,
        out_shape=jax.ShapeDtypeStruct((M, H), jnp.float32),
    )(g1, g2, wc, b)


def _final(C, D, he, te, bm=256, bn=512):
    M = C.shape[0]
    N2 = D.shape[0]
    dn = (((1,), (1,)), ((), ()))

    def body(c_ref, d_ref, h_ref, t_ref, o_ref):
        o_ref[...] = (lax.dot_general(c_ref[...], d_ref[...], dn,
                                      preferred_element_type=jnp.float32)
                      + lax.dot_general(h_ref[...], t_ref[...], dn,
                                        preferred_element_type=jnp.float32))

    return pl.pallas_call(
        body,
        grid=(M // bm, N2 // bn),
        in_specs=[pl.BlockSpec((bm, H), lambda i, j: (i, 0)),
                  pl.BlockSpec((bn, H), lambda i, j: (j, 0)),
                  pl.BlockSpec((bm, H), lambda i, j: (i, 0)),
                  pl.BlockSpec((bn, H), lambda i, j: (j, 0))],
        out_specs=pl.BlockSpec((bm, bn), lambda i, j: (i, j)),
        out_shape=jax.ShapeDtypeStruct((M, N2), jnp.float32),
    )(C, D, he, te)


def _pack_sd(src, dst, ch):
    return jnp.concatenate(
        [src.reshape(-1, 1, ch), dst.reshape(-1, 1, ch)],
        axis=1).reshape(-1)


def _branch(x, ei, p, pre):
    n = x.shape[0]
    npad = n + 128
    e = ei.shape[1]
    ep = e + n
    epad = -(-ep // 8192) * 8192
    loop = jnp.arange(n, dtype=jnp.int32)
    src = jnp.concatenate([ei[0].astype(jnp.int32), loop,
                           jnp.zeros((epad - ep,), jnp.int32)])
    dst = jnp.concatenate([ei[1].astype(jnp.int32), loop,
                           jnp.full((epad - ep,), n, jnp.int32)])
    sd_a = _pack_sd(src, dst, 256)
    sd_b = _pack_sd(src, dst, 128)

    degs = _sc_hist(dst, npad)
    dis = _dis_kernel(degs)
    disn = dis[:n]

    h0 = _matmul_bias(x, p[pre + "_proj_W"], p[pre + "_proj_b"][None])

    t1 = _scale_matmul(h0, p[pre + "_gcn1_W"], disn)
    a1 = _sc_gs(t1.reshape(n * 8, 16), sd_a, npad, epad)
    g1 = _combine_gs(a1, disn, p[pre + "_gcn1_b"][None], h0, n)

    xw4, als, ald, smax = _gat_lin(g1, p[pre + "_gat_W"],
                                   p[pre + "_gat_as"], p[pre + "_gat_ad"])
    rowpad = jnp.zeros((4, npad - n), jnp.float32)
    als_t = jnp.concatenate([als.T, rowpad], axis=1)
    ald_t = jnp.concatenate([ald.T, rowpad], axis=1)
    smax16 = jnp.pad(smax.reshape(-1), (0, LANES - 4))
    q, zout = _sc_den(sd_b, als_t, ald_t, smax16, npad, epad)
    sel = ((jnp.arange(NT * LANES)[:, None] // LANES) % 4
           == jnp.arange(4)[None, :]).astype(jnp.float32) / LANES
    inv = _postden(zout, sel)
    w4 = _sc_att(q, dst, inv.reshape(-1), npad)
    a2 = _sc_gsw(xw4.reshape(n * 32, 16), sd_b, w4, npad, epad)
    ga = _combine_gat(a2, p[pre + "_gat_b"][None], g1, n)

    t2 = _scale_matmul(ga, p[pre + "_gcn2_W"], disn)
    a3 = _sc_gs(t2.reshape(n * 8, 16), sd_a, npad, epad)
    g2 = _combine_gs(a3, disn, p[pre + "_gcn2_b"][None], ga, n)

    return _cnn(g1, g2, p[pre + "_cnn_W"], p[pre + "_cnn_b"][None])


def kernel(het_net, het_x, herb_net, herb_x, target_net, target_x, params):
    p = params
    het_e = _branch(het_x, het_net, p, "het")
    herb_e = _branch(herb_x, herb_net, p, "herb")
    tgt_e = _branch(target_x, target_net, p, "tgt")
    C = het_e[:N_HERB]
    D = het_e[N_HERB:]
    return _final(C, D, herb_e, tgt_e)


# speedup vs baseline: 16.4817x; 1.0450x over previous
import dataclasses
import functools

import jax
import jax.numpy as jnp
from jax import lax
from jax.experimental import pallas as pl
from jax.experimental.pallas import tpu as pltpu
from jax.experimental.pallas import tpu_sc as plsc

H = 128
N_HERB = 2048
NC = 2
NS = 16
NT = NC * NS
LANES = 16


def _mesh():
    return plsc.VectorSubcoreMesh(core_axis_name="c", subcore_axis_name="s")


def _sc_params():
    cp = pltpu.CompilerParams()
    fields = pltpu.CompilerParams.__dataclass_fields__
    if "needs_layout_passes" in fields:
        cp = dataclasses.replace(cp, needs_layout_passes=False)
    if "use_tc_tiling_on_sc" in fields:
        cp = dataclasses.replace(cp, use_tc_tiling_on_sc=False)
    return cp


def _zero_rows(buf):
    R, W = buf.shape
    z = jnp.zeros((LANES,), jnp.float32)

    @pl.loop(0, R)
    def _(i):
        buf[i, :] = z


def _sc_hist(dst, npad):
    (E,) = dst.shape
    CH = 256
    cpt = E // (NT * CH)

    @functools.partial(
        pl.kernel,
        out_type=jax.ShapeDtypeStruct((npad, NT * LANES), jnp.float32),
        mesh=_mesh(),
        compiler_params=_sc_params(),
        scratch_types=[
            pltpu.VMEM((CH,), jnp.int32),
            pltpu.VMEM((npad, LANES), jnp.float32),
        ],
    )
    def k(dst_hbm, out_hbm, d_v, acc_v):
        cid = lax.axis_index("c")
        sid = lax.axis_index("s")
        t = sid * NC + cid
        _zero_rows(acc_v)
        iota = lax.iota(jnp.int32, LANES)
        ones = jnp.full((LANES,), 1.0, jnp.float32)

        @pl.loop(0, cpt)
        def _(j):
            base = (t * cpt + j) * CH
            pltpu.sync_copy(dst_hbm.at[pl.ds(base, CH)], d_v)

            @plsc.parallel_loop(0, CH, unroll=8)
            def _(e):
                dsp = plsc.load_gather(d_v, [jnp.full((LANES,), e,
                                                      jnp.int32)])
                plsc.addupdate_scatter(acc_v, [dsp, iota], ones)

        pltpu.sync_copy(acc_v, out_hbm.at[:, pl.ds(t * LANES, LANES)])

    return k(dst)


def _sc_gs(tab8, sd, npad, E):
    CH = 256
    NG = 4
    cpt = E // (NG * CH)
    half = cpt // 2

    @functools.partial(
        pl.kernel,
        out_type=jax.ShapeDtypeStruct((NG, npad, H), jnp.float32),
        mesh=_mesh(),
        compiler_params=_sc_params(),
        scratch_types=[
            pltpu.VMEM((2 * CH,), jnp.int32),
            pltpu.VMEM((2 * CH,), jnp.int32),
            pltpu.VMEM((CH,), jnp.int32),
            pltpu.VMEM((CH,), jnp.int32),
            pltpu.VMEM((CH,), jnp.int32),
            pltpu.VMEM((CH,), jnp.int32),
            pltpu.VMEM((CH, LANES), jnp.float32),
            pltpu.VMEM((CH, LANES), jnp.float32),
            pltpu.VMEM((npad, LANES), jnp.float32),
            pltpu.SemaphoreType.DMA,
            pltpu.SemaphoreType.DMA,
            pltpu.SemaphoreType.DMA,
            pltpu.SemaphoreType.DMA,
        ],
    )
    def k(tab_hbm, sd_hbm, out_hbm, sd0, sd1, gi0, gi1, dv0, dv1, rw0, rw1,
          acc_v, sm0, sm1, is0, is1):
        cid = lax.axis_index("c")
        sid = lax.axis_index("s")
        t = sid * NC + cid
        g = t // 8
        cs = t - g * 8
        _zero_rows(acc_v)
        iota = lax.iota(jnp.int32, LANES)
        sdv = (sd0, sd1)
        giv = (gi0, gi1)
        dvv = (dv0, dv1)
        rwv = (rw0, rw1)
        smv = (sm0, sm1)
        ism = (is0, is1)

        def start_idx(b, j):
            base = (g * cpt + j) * CH
            pltpu.make_async_copy(sd_hbm.at[pl.ds(base * 2, 2 * CH)],
                                  sdv[b], ism[b]).start()

        def arm(b):
            pltpu.make_async_copy(sd_hbm.at[pl.ds(0, 2 * CH)], sdv[b],
                                  ism[b]).wait()

            @pl.loop(0, CH, step=LANES)
            def _(o):
                giv[b][pl.ds(o, LANES)] = sdv[b][pl.ds(o, LANES)] * 8 + cs
                dvv[b][pl.ds(o, LANES)] = sdv[b][pl.ds(CH + o, LANES)]

            pltpu.make_async_copy(tab_hbm.at[giv[b]], rwv[b],
                                  smv[b]).start()

        def fin(b):
            pltpu.make_async_copy(tab_hbm.at[giv[b]], rwv[b],
                                  smv[b]).wait()

            @plsc.parallel_loop(0, CH, unroll=8)
            def _(e):
                dsp = plsc.load_gather(
                    dvv[b], [jnp.full((LANES,), e, jnp.int32)])
                plsc.addupdate_scatter(acc_v, [dsp, iota], rwv[b][e, :])

        start_idx(0, 0)
        start_idx(1, 1)
        arm(0)

        @pl.loop(0, half)
        def _(j2):
            j = j2 * 2
            arm(1)
            start_idx(0, jnp.minimum(j + 2, cpt - 1))
            fin(0)
            arm(0)
            start_idx(1, jnp.minimum(j + 3, cpt - 1))
            fin(1)

        pltpu.make_async_copy(tab_hbm.at[giv[0]], rwv[0], smv[0]).wait()
        pltpu.make_async_copy(sd_hbm.at[pl.ds(0, 2 * CH)], sdv[1],
                              ism[1]).wait()
        pltpu.sync_copy(acc_v, out_hbm.at[g, :, pl.ds(cs * LANES, LANES)])

    return k(tab8, sd)


def _sc_den(sd, als_t, ald_t, smax16, npad, E):
    CH = 128
    G = CH // LANES
    NG = 8
    cpt = E // (NG * CH)

    @functools.partial(
        pl.kernel,
        out_type=(jax.ShapeDtypeStruct((4, E), jnp.float32),
                  jax.ShapeDtypeStruct((npad, NT * LANES), jnp.float32)),
        mesh=_mesh(),
        compiler_params=_sc_params(),
        scratch_types=[
            pltpu.VMEM((2 * CH,), jnp.int32),
            pltpu.VMEM((2 * CH,), jnp.int32),
            pltpu.VMEM((npad,), jnp.float32),
            pltpu.VMEM((npad,), jnp.float32),
            pltpu.VMEM((LANES,), jnp.float32),
            pltpu.VMEM((CH,), jnp.float32),
            pltpu.VMEM((CH,), jnp.float32),
            pltpu.VMEM((npad, LANES), jnp.float32),
            pltpu.SemaphoreType.DMA,
            pltpu.SemaphoreType.DMA,
            pltpu.SemaphoreType.DMA,
            pltpu.SemaphoreType.DMA,
        ],
    )
    def k(sd_hbm, als_hbm, ald_hbm, sm_hbm, q_hbm, z_hbm,
          sd0, sd1, als_v, ald_v, sm_v, qb0, qb1, acc_v,
          is0, is1, qs0, qs1):
        cid = lax.axis_index("c")
        sid = lax.axis_index("s")
        t = sid * NC + cid
        g = t // 4
        h = t - g * 4
        pltpu.sync_copy(als_hbm.at[h], als_v)
        pltpu.sync_copy(ald_hbm.at[h], ald_v)
        pltpu.sync_copy(sm_hbm, sm_v)
        _zero_rows(acc_v)
        iota = lax.iota(jnp.int32, LANES)
        smsp = plsc.load_gather(sm_v, [jnp.full((LANES,), h, jnp.int32)])
        half = cpt // 2
        sdv = (sd0, sd1)
        qbv = (qb0, qb1)
        ism = (is0, is1)
        qsm = (qs0, qs1)

        def start_idx(b, j):
            base = (g * cpt + j) * CH
            pltpu.make_async_copy(sd_hbm.at[pl.ds(base * 2, 2 * CH)],
                                  sdv[b], ism[b]).start()

        def proc(b, j, first):
            base = (g * cpt + j) * CH
            pltpu.make_async_copy(sd_hbm.at[pl.ds(0, 2 * CH)], sdv[b],
                                  ism[b]).wait()

            @pl.when(jnp.logical_not(first))
            def _():
                pltpu.make_async_copy(qbv[b], q_hbm.at[h, pl.ds(0, CH)],
                                      qsm[b]).wait()

            @plsc.parallel_loop(0, G, unroll=4)
            def _(gg):
                sl = pl.ds(gg * LANES, LANES)
                av = plsc.load_gather(als_v, [sdv[b][sl]])
                dv = plsc.load_gather(
                    ald_v, [sdv[b][pl.ds(CH + gg * LANES, LANES)]])
                al = av + dv
                al = jnp.maximum(al, al * 0.2)
                cc = dv + smsp
                cc = jnp.maximum(cc, cc * 0.2)
                qbv[b][sl] = jnp.exp(al - cc)

            @plsc.parallel_loop(0, CH, unroll=8)
            def _(e):
                ee = jnp.full((LANES,), e, jnp.int32)
                dsp = plsc.load_gather(sdv[b], [ee + CH])
                qsp = plsc.load_gather(qbv[b], [ee])
                plsc.addupdate_scatter(acc_v, [dsp, iota], qsp)

            pltpu.make_async_copy(qbv[b], q_hbm.at[h, pl.ds(base, CH)],
                                  qsm[b]).start()

        start_idx(0, 0)
        start_idx(1, 1)

        @pl.loop(0, half)
        def _(j2):
            j = j2 * 2
            proc(0, j, j2 == 0)
            start_idx(0, jnp.minimum(j + 2, cpt - 1))
            proc(1, j + 1, j2 == 0)
            start_idx(1, jnp.minimum(j + 3, cpt - 1))

        for b in range(2):
            pltpu.make_async_copy(qbv[b], q_hbm.at[h, pl.ds(0, CH)],
                                  qsm[b]).wait()
            pltpu.make_async_copy(sd_hbm.at[pl.ds(0, 2 * CH)], sdv[b],
                                  ism[b]).wait()
        pltpu.sync_copy(acc_v, z_hbm.at[:, pl.ds(t * LANES, LANES)])

    return k(sd, als_t, ald_t, smax16)


def _sc_att(q, dst, inv, npad):
    four, E = q.shape
    CH = 128
    G = CH // LANES
    cpt = E // (NT * CH)

    @functools.partial(
        pl.kernel,
        out_type=jax.ShapeDtypeStruct((4, E), jnp.float32),
        mesh=_mesh(),
        compiler_params=_sc_params(),
        scratch_types=[
            pltpu.VMEM((CH,), jnp.int32),
            pltpu.VMEM((4, CH), jnp.float32),
            pltpu.VMEM((npad * 4,), jnp.float32),
            pltpu.SemaphoreType.DMA,
        ],
    )
    def k(q_hbm, dst_hbm, inv_hbm, w_hbm, d_v, q_v, inv_v, sem):
        cid = lax.axis_index("c")
        sid = lax.axis_index("s")
        t = sid * NC + cid
        pltpu.sync_copy(inv_hbm, inv_v)

        @pl.loop(0, cpt)
        def _(j):
            base = (t * cpt + j) * CH
            pltpu.sync_copy(dst_hbm.at[pl.ds(base, CH)], d_v)
            for h in range(4):
                pltpu.make_async_copy(q_hbm.at[h, pl.ds(base, CH)],
                                      q_v.at[h], sem).start()
            for h in range(4):
                pltpu.make_async_copy(q_hbm.at[h, pl.ds(base, CH)],
                                      q_v.at[h], sem).wait()
            for h in range(4):
                @plsc.parallel_loop(0, G, unroll=4)
                def _(gg):
                    sl = pl.ds(gg * LANES, LANES)
                    iv = plsc.load_gather(inv_v, [d_v[sl] * 4 + h])
                    q_v[h, sl] = q_v[h, sl] * iv

            for h in range(4):
                pltpu.sync_copy(q_v.at[h], w_hbm.at[h, pl.ds(base, CH)])

    return k(q, dst, inv)


def _sc_gsw(tab32, sd, w4, npad, E):
    CH = 128
    NG = 4
    cpt = E // (NG * CH)
    half = cpt // 2

    @functools.partial(
        pl.kernel,
        out_type=jax.ShapeDtypeStruct((NG, npad, H), jnp.float32),
        mesh=_mesh(),
        compiler_params=_sc_params(),
        scratch_types=[
            pltpu.VMEM((2 * CH,), jnp.int32),
            pltpu.VMEM((2 * CH,), jnp.int32),
            pltpu.VMEM((4, CH), jnp.int32),
            pltpu.VMEM((4, CH), jnp.int32),
            pltpu.VMEM((CH,), jnp.int32),
            pltpu.VMEM((CH,), jnp.int32),
            pltpu.VMEM((4 * CH,), jnp.float32),
            pltpu.VMEM((4 * CH,), jnp.float32),
            pltpu.VMEM((4, CH, LANES), jnp.float32),
            pltpu.VMEM((4, CH, LANES), jnp.float32),
            pltpu.VMEM((npad, LANES), jnp.float32),
            pltpu.SemaphoreType.DMA,
            pltpu.SemaphoreType.DMA,
            pltpu.SemaphoreType.DMA,
            pltpu.SemaphoreType.DMA,
        ],
    )
    def k(tab_hbm, sd_hbm, w_hbm, out_hbm, sd0, sd1, gi0, gi1, dv0, dv1,
          wb0, wb1, rw0, rw1, acc_v, sm0, sm1, is0, is1):
        cid = lax.axis_index("c")
        sid = lax.axis_index("s")
        t = sid * NC + cid
        g = t // 8
        cs = t - g * 8
        _zero_rows(acc_v)
        iota = lax.iota(jnp.int32, LANES)
        sdv = (sd0, sd1)
        giv = (gi0, gi1)
        dvv = (dv0, dv1)
        wbv = (wb0, wb1)
        rwv = (rw0, rw1)
        smv = (sm0, sm1)
        ism = (is0, is1)

        def start_idx(b, j):
            base = (g * cpt + j) * CH
            pltpu.make_async_copy(sd_hbm.at[pl.ds(base * 2, 2 * CH)],
                                  sdv[b], ism[b]).start()

        def arm(b, j):
            base = (g * cpt + j) * CH
            pltpu.make_async_copy(sd_hbm.at[pl.ds(0, 2 * CH)], sdv[b],
                                  ism[b]).wait()
            for h in range(4):
                pltpu.make_async_copy(w_hbm.at[h, pl.ds(base, CH)],
                                      wbv[b].at[pl.ds(h * CH, CH)],
                                      smv[b]).start()

                @pl.loop(0, CH, step=LANES)
                def _(o):
                    giv[b][h, pl.ds(o, LANES)] = (
                        sdv[b][pl.ds(o, LANES)] * 32 + (h * 8) + cs)

                pltpu.make_async_copy(tab_hbm.at[giv[b].at[h]],
                                     rwv[b].at[h], smv[b]).start()

            @pl.loop(0, CH, step=LANES)
            def _(o):
                dvv[b][pl.ds(o, LANES)] = sdv[b][pl.ds(CH + o, LANES)]

        def wait_data(b):
            for h in range(4):
                pltpu.make_async_copy(tab_hbm.at[giv[b].at[h]],
                                      rwv[b].at[h], smv[b]).wait()
                pltpu.make_async_copy(w_hbm.at[h, pl.ds(0, CH)],
                                      wbv[b].at[pl.ds(h * CH, CH)],
                                      smv[b]).wait()

        def fin(b):
            wait_data(b)

            @plsc.parallel_loop(0, CH, unroll=4)
            def _(e):
                ee = jnp.full((LANES,), e, jnp.int32)
                dsp = plsc.load_gather(dvv[b], [ee])
                w0 = plsc.load_gather(wbv[b], [ee])
                acc = rwv[b][0, e, :] * w0
                for h in range(1, 4):
                    wh = plsc.load_gather(wbv[b], [h * CH + ee])
                    acc = acc + rwv[b][h, e, :] * wh
                plsc.addupdate_scatter(acc_v, [dsp, iota], acc)

        start_idx(0, 0)
        start_idx(1, 1)
        arm(0, 0)

        @pl.loop(0, half)
        def _(j2):
            j = j2 * 2
            arm(1, j + 1)
            start_idx(0, jnp.minimum(j + 2, cpt - 1))
            fin(0)
            arm(0, jnp.minimum(j + 2, cpt - 1))
            start_idx(1, jnp.minimum(j + 3, cpt - 1))
            fin(1)

        wait_data(0)
        pltpu.make_async_copy(sd_hbm.at[pl.ds(0, 2 * CH)], sdv[1],
                              ism[1]).wait()
        pltpu.sync_copy(acc_v, out_hbm.at[g, :, pl.ds(cs * LANES, LANES)])

    return k(tab32, sd, w4)


def _matmul_bias(x, w, b, bm=256, bk=512):
    M, K = x.shape
    _, Ho = w.shape
    bk = min(bk, K)
    grid = (M // bm, K // bk)

    def body(x_ref, w_ref, b_ref, o_ref):
        kk = pl.program_id(1)

        @pl.when(kk == 0)
        def _():
            o_ref[...] = jnp.broadcast_to(b_ref[...], o_ref.shape)

        o_ref[...] += jnp.dot(x_ref[...], w_ref[...],
                              preferred_element_type=jnp.float32)

    return pl.pallas_call(
        body,
        grid=grid,
        in_specs=[pl.BlockSpec((bm, bk), lambda i, k: (i, k)),
                  pl.BlockSpec((bk, Ho), lambda i, k: (k, 0)),
                  pl.BlockSpec((1, Ho), lambda i, k: (0, 0))],
        out_specs=pl.BlockSpec((bm, Ho), lambda i, k: (i, 0)),
        out_shape=jax.ShapeDtypeStruct((M, Ho), jnp.float32),
    )(x, w, b)


def _scale_matmul(x, w, s, bm=512):
    M, K = x.shape
    _, Ho = w.shape

    def body(x_ref, w_ref, s_ref, o_ref):
        o_ref[...] = jnp.dot(x_ref[...], w_ref[...],
                             preferred_element_type=jnp.float32) * s_ref[...]

    return pl.pallas_call(
        body,
        grid=(M // bm,),
        in_specs=[pl.BlockSpec((bm, K), lambda i: (i, 0)),
                  pl.BlockSpec((K, Ho), lambda i: (0, 0)),
                  pl.BlockSpec((bm, 1), lambda i: (i, 0))],
        out_specs=pl.BlockSpec((bm, Ho), lambda i: (i, 0)),
        out_shape=jax.ShapeDtypeStruct((M, Ho), jnp.float32),
    )(x, w, s)


def _gat_lin(g1, wg, a_s, a_d, bm=512):
    M, K = g1.shape
    _, W4 = wg.shape

    def body(g_ref, w_ref, as_ref, ad_ref, xw_ref, als_ref, ald_ref,
             sm_ref):
        i = pl.program_id(0)
        xw = jnp.dot(g_ref[...], w_ref[...],
                     preferred_element_type=jnp.float32)
        xw_ref[...] = xw
        als_cols = []
        ald_cols = []
        for h in range(4):
            sl = xw[:, h * H:(h + 1) * H]
            als_cols.append(jnp.sum(sl * as_ref[h:h + 1, :], axis=1,
                                    keepdims=True))
            ald_cols.append(jnp.sum(sl * ad_ref[h:h + 1, :], axis=1,
                                    keepdims=True))
        als_blk = jnp.concatenate(als_cols, axis=1)
        ald_blk = jnp.concatenate(ald_cols, axis=1)
        als_ref[...] = als_blk
        ald_ref[...] = ald_blk
        loc = jnp.max(als_blk, axis=0, keepdims=True)

        @pl.when(i == 0)
        def _():
            sm_ref[...] = loc

        @pl.when(i > 0)
        def _():
            sm_ref[...] = jnp.maximum(sm_ref[...], loc)

    return pl.pallas_call(
        body,
        grid=(M // bm,),
        in_specs=[pl.BlockSpec((bm, K), lambda i: (i, 0)),
                  pl.BlockSpec((K, W4), lambda i: (0, 0)),
                  pl.BlockSpec((4, H), lambda i: (0, 0)),
                  pl.BlockSpec((4, H), lambda i: (0, 0))],
        out_specs=[pl.BlockSpec((bm, W4), lambda i: (i, 0)),
                   pl.BlockSpec((bm, 4), lambda i: (i, 0)),
                   pl.BlockSpec((bm, 4), lambda i: (i, 0)),
                   pl.BlockSpec((1, 4), lambda i: (0, 0))],
        out_shape=[jax.ShapeDtypeStruct((M, W4), jnp.float32),
                   jax.ShapeDtypeStruct((M, 4), jnp.float32),
                   jax.ShapeDtypeStruct((M, 4), jnp.float32),
                   jax.ShapeDtypeStruct((1, 4), jnp.float32)],
    )(g1, wg, a_s, a_d)


def _dis_kernel(degs):
    npad = degs.shape[0]

    def body(p_ref, o_ref):
        deg = jnp.sum(p_ref[...], axis=1, keepdims=True) * (1.0 / LANES)
        o_ref[...] = jnp.where(deg > 0, lax.rsqrt(deg), 0.0)

    return pl.pallas_call(
        body,
        out_shape=jax.ShapeDtypeStruct((npad, 1), jnp.float32),
    )(degs)


def _postden(zout, sel):
    npad = zout.shape[0]

    def body(p_ref, s_ref, o_ref):
        z = jnp.dot(p_ref[...], s_ref[...],
                    preferred_element_type=jnp.float32)
        o_ref[...] = 0.25 / (z + 1e-16)

    return pl.pallas_call(
        body,
        out_shape=jax.ShapeDtypeStruct((npad, 4), jnp.float32),
    )(zout, sel)


def _combine_gs(parts, dis, b, res, n):
    def body(p_ref, d_ref, b_ref, r_ref, o_ref):
        x = jnp.sum(p_ref[...], axis=0)[:n]
        o_ref[...] = jnp.maximum(x * d_ref[...] + b_ref[...] + r_ref[...],
                                 0.0)

    return pl.pallas_call(
        body,
        out_shape=jax.ShapeDtypeStruct((n, H), jnp.float32),
    )(parts, dis, b, res)


def _combine_gat(parts, b, res, n):
    def body(p_ref, b_ref, r_ref, o_ref):
        x = jnp.sum(p_ref[...], axis=0)[:n]
        o_ref[...] = jnp.maximum(x + b_ref[...] + r_ref[...], 0.0)

    return pl.pallas_call(
        body,
        out_shape=jax.ShapeDtypeStruct((n, H), jnp.float32),
    )(parts, b, res)


def _cnn(g1, g2, wc, b):
    M = g1.shape[0]
    dn = (((1,), (1,)), ((), ()))

    def body(g1_ref, g2_ref, w_ref, b_ref, o_ref):
        w0 = w_ref[:, 0, :]
        w1 = w_ref[:, 1, :]
        o_ref[...] = (lax.dot_general(g1_ref[...], w0, dn,
                                      preferred_element_type=jnp.float32)
                      + lax.dot_general(g2_ref[...], w1, dn,
                                        preferred_element_type=jnp.float32)
                      + b_ref[...])

    return pl.pallas_call(
        body,
        out_shape=jax.ShapeDtypeStruct((M, H), jnp.float32),
    )(g1, g2, wc, b)


def _final(C, D, he, te, bm=256, bn=512):
    M = C.shape[0]
    N2 = D.shape[0]
    dn = (((1,), (1,)), ((), ()))

    def body(c_ref, d_ref, h_ref, t_ref, o_ref):
        o_ref[...] = (lax.dot_general(c_ref[...], d_ref[...], dn,
                                      preferred_element_type=jnp.float32)
                      + lax.dot_general(h_ref[...], t_ref[...], dn,
                                        preferred_element_type=jnp.float32))

    return pl.pallas_call(
        body,
        grid=(M // bm, N2 // bn),
        in_specs=[pl.BlockSpec((bm, H), lambda i, j: (i, 0)),
                  pl.BlockSpec((bn, H), lambda i, j: (j, 0)),
                  pl.BlockSpec((bm, H), lambda i, j: (i, 0)),
                  pl.BlockSpec((bn, H), lambda i, j: (j, 0))],
        out_specs=pl.BlockSpec((bm, bn), lambda i, j: (i, j)),
        out_shape=jax.ShapeDtypeStruct((M, N2), jnp.float32),
    )(C, D, he, te)


def _pack_sd(src, dst, ch):
    return jnp.concatenate(
        [src.reshape(-1, 1, ch), dst.reshape(-1, 1, ch)],
        axis=1).reshape(-1)


def _branch(x, ei, p, pre):
    n = x.shape[0]
    npad = n + 128
    e = ei.shape[1]
    ep = e + n
    epad = -(-ep // 8192) * 8192
    loop = jnp.arange(n, dtype=jnp.int32)
    src = jnp.concatenate([ei[0].astype(jnp.int32), loop,
                           jnp.zeros((epad - ep,), jnp.int32)])
    dst = jnp.concatenate([ei[1].astype(jnp.int32), loop,
                           jnp.full((epad - ep,), n, jnp.int32)])
    sd_a = _pack_sd(src, dst, 256)
    sd_b = _pack_sd(src, dst, 128)

    degs = _sc_hist(dst, npad)
    dis = _dis_kernel(degs)
    disn = dis[:n]

    h0 = _matmul_bias(x, p[pre + "_proj_W"], p[pre + "_proj_b"][None])

    t1 = _scale_matmul(h0, p[pre + "_gcn1_W"], disn)
    a1 = _sc_gs(t1.reshape(n * 8, 16), sd_a, npad, epad)
    g1 = _combine_gs(a1, disn, p[pre + "_gcn1_b"][None], h0, n)

    xw4, als, ald, smax = _gat_lin(g1, p[pre + "_gat_W"],
                                   p[pre + "_gat_as"], p[pre + "_gat_ad"])
    rowpad = jnp.zeros((4, npad - n), jnp.float32)
    als_t = jnp.concatenate([als.T, rowpad], axis=1)
    ald_t = jnp.concatenate([ald.T, rowpad], axis=1)
    smax16 = jnp.pad(smax.reshape(-1), (0, LANES - 4))
    q, zout = _sc_den(sd_b, als_t, ald_t, smax16, npad, epad)
    sel = ((jnp.arange(NT * LANES)[:, None] // LANES) % 4
           == jnp.arange(4)[None, :]).astype(jnp.float32) / LANES
    inv = _postden(zout, sel)
    w4 = _sc_att(q, dst, inv.reshape(-1), npad)
    a2 = _sc_gsw(xw4.reshape(n * 32, 16), sd_b, w4, npad, epad)
    ga = _combine_gat(a2, p[pre + "_gat_b"][None], g1, n)

    t2 = _scale_matmul(ga, p[pre + "_gcn2_W"], disn)
    a3 = _sc_gs(t2.reshape(n * 8, 16), sd_a, npad, epad)
    g2 = _combine_gs(a3, disn, p[pre + "_gcn2_b"][None], ga, n)

    return _cnn(g1, g2, p[pre + "_cnn_W"], p[pre + "_cnn_b"][None])


def kernel(het_net, het_x, herb_net, herb_x, target_net, target_x, params):
    p = params
    het_e = _branch(het_x, het_net, p, "het")
    herb_e = _branch(herb_x, herb_net, p, "herb")
    tgt_e = _branch(target_x, target_net, p, "tgt")
    C = het_e[:N_HERB]
    D = het_e[N_HERB:]
    return _final(C, D, herb_e, tgt_e)
